# Initial kernel scaffold; baseline (speedup 1.0000x reference)
import jax
import jax.numpy as jnp
from jax.experimental import pallas as pl


def kernel(x, style_vector, edge_index, edge_attr, batch_size, nroi, params):
    raise NotImplementedError("write your pallas kernel here")


# speedup vs baseline: 5.3673x; 5.3673x over previous
import functools

import jax
import jax.numpy as jnp
from jax import lax
from jax.experimental import pallas as pl
from jax.experimental.pallas import tpu as pltpu
from jax.experimental.pallas import tpu_sc as plsc

_NC = 2
_NS = 16
_NPAD = 10240
_CH = 80


def _leaky(v):
    return jnp.where(v > 0, v, 0.2 * v)


def _dis_of(deg):
    return jnp.where(deg > 0, 1.0 / jnp.sqrt(deg + 1e-12), 0.0)


def _deg_body(src_hbm, ew_hbm, deg0_hbm, deg1_hbm, idx_v, ew_v, z_v, acc_sh, sem):
    c = lax.axis_index("c")
    s = lax.axis_index("s")
    E = src_hbm.shape[0]
    half = E // _NC
    per_tile = half // _NS
    nch = per_tile // _CH
    sl = _NPAD // _NS

    def zb(i, _):
        z_v[pl.ds(i * 16, 16)] = jnp.zeros((16,), jnp.float32)
        return 0
    lax.fori_loop(0, sl // 16, zb, 0)
    pltpu.sync_copy(z_v, acc_sh.at[pl.ds(s * sl, sl)])
    plsc.subcore_barrier()

    base = c * half + s * per_tile

    def body(i, _):
        off = base + i * _CH
        pltpu.sync_copy(src_hbm.at[pl.ds(off, _CH)], idx_v)
        pltpu.sync_copy(ew_hbm.at[pl.ds(off, _CH)], ew_v)
        pltpu.sync_copy(ew_v, acc_sh.at[idx_v], add=True)
        return 0
    lax.fori_loop(0, nch, body, 0)
    plsc.subcore_barrier()

    @pl.when(c == 0)
    def _():
        pltpu.sync_copy(acc_sh.at[pl.ds(s * sl, sl)], deg0_hbm.at[pl.ds(s * sl, sl)])

    @pl.when(c == 1)
    def _():
        pltpu.sync_copy(acc_sh.at[pl.ds(s * sl, sl)], deg1_hbm.at[pl.ds(s * sl, sl)])


def _deg_call(src, ew):
    mesh = plsc.VectorSubcoreMesh(core_axis_name="c", subcore_axis_name="s")
    f = pl.kernel(
        _deg_body,
        out_type=[jax.ShapeDtypeStruct((_NPAD,), jnp.float32),
                  jax.ShapeDtypeStruct((_NPAD,), jnp.float32)],
        mesh=mesh,
        scratch_types=[
            pltpu.VMEM((_CH,), jnp.int32),
            pltpu.VMEM((_CH,), jnp.float32),
            pltpu.VMEM((_NPAD // _NS,), jnp.float32),
            pltpu.VMEM_SHARED((_NPAD,), jnp.float32),
            pltpu.SemaphoreType.DMA,
        ],
    )
    return f(src, ew)


def _acc1_body(u0, u1, u2, u3, src_hbm, dst_hbm, ew_hbm,
               a0, a1, a2, a3,
               sidx, didx, ew_v, rows, zrows, acc_sh, sem):
    c = lax.axis_index("c")
    s = lax.axis_index("s")
    E = src_hbm.shape[0]
    per_tile = E // _NS
    nch = per_tile // _CH
    sl = _NPAD // _NS
    nsub = sl // _CH

    def zb(k, _):
        for j in range(8):
            zrows[k, pl.ds(j * 16, 16)] = jnp.zeros((16,), jnp.float32)
        return 0
    lax.fori_loop(0, _CH, zb, 0)

    u_all = (u0, u1, u2, u3)
    out_all = (a0, a1, a2, a3)
    for p in range(4):
        @pl.when(c == p // 2)
        def _(p=p):
            u_hbm = u_all[p]
            out_hbm = out_all[p]

            def z(i, _):
                pltpu.sync_copy(zrows, acc_sh.at[pl.ds(s * sl + i * _CH, _CH)])
                return 0
            lax.fori_loop(0, nsub, z, 0)
            plsc.subcore_barrier()

            base = s * per_tile

            def body(i, _):
                off = base + i * _CH
                pltpu.sync_copy(src_hbm.at[pl.ds(off, _CH)], sidx)
                pltpu.sync_copy(dst_hbm.at[pl.ds(off, _CH)], didx)
                pltpu.sync_copy(ew_hbm.at[pl.ds(off, _CH)], ew_v)
                pltpu.async_copy(u_hbm.at[sidx], rows, sem).wait()

                def scale(g, _):
                    w16 = ew_v[pl.ds(g * 16, 16)]
                    for l in range(16):
                        w = w16[l]
                        k = g * 16 + l
                        for j in range(8):
                            rows[k, pl.ds(j * 16, 16)] = rows[k, pl.ds(j * 16, 16)] * w
                    return 0
                lax.fori_loop(0, _CH // 16, scale, 0)
                pltpu.sync_copy(rows, acc_sh.at[didx], add=True)
                return 0
            lax.fori_loop(0, nch, body, 0)
            plsc.subcore_barrier()

            def wout(i, _):
                pltpu.sync_copy(acc_sh.at[pl.ds(s * sl + i * _CH, _CH)], rows)
                pltpu.sync_copy(rows, out_hbm.at[pl.ds(s * sl + i * _CH, _CH)])
                return 0
            lax.fori_loop(0, nsub, wout, 0)
            plsc.subcore_barrier()


def _acc1_call(u_panels, src, dst, ew):
    mesh = plsc.VectorSubcoreMesh(core_axis_name="c", subcore_axis_name="s")
    f = pl.kernel(
        _acc1_body,
        out_type=[jax.ShapeDtypeStruct((_NPAD, 128), jnp.float32)] * 4,
        mesh=mesh,
        scratch_types=[
            pltpu.VMEM((_CH,), jnp.int32),
            pltpu.VMEM((_CH,), jnp.int32),
            pltpu.VMEM((_CH,), jnp.float32),
            pltpu.VMEM((_CH, 128), jnp.float32),
            pltpu.VMEM((_CH, 128), jnp.float32),
            pltpu.VMEM_SHARED((_NPAD, 128), jnp.float32),
            pltpu.SemaphoreType.DMA,
        ],
    )
    return f(*u_panels, src, dst, ew)


def _acc2_body(u_hbm, src_hbm, dst_hbm, ew_hbm, b0, b1,
               sidx, didx, ew_v, rows, zrows, acc_sh, sem):
    c = lax.axis_index("c")
    s = lax.axis_index("s")
    E = src_hbm.shape[0]
    half = E // _NC
    per_tile = half // _NS
    nch = per_tile // _CH
    sl = _NPAD // _NS
    nsub = sl // _CH

    def zb(k, _):
        for j in range(8):
            zrows[k, pl.ds(j * 16, 16)] = jnp.zeros((16,), jnp.float32)
        return 0
    lax.fori_loop(0, _CH, zb, 0)

    def z(i, _):
        pltpu.sync_copy(zrows, acc_sh.at[pl.ds(s * sl + i * _CH, _CH)])
        return 0
    lax.fori_loop(0, nsub, z, 0)
    plsc.subcore_barrier()

    base = c * half + s * per_tile

    def body(i, _):
        off = base + i * _CH
        pltpu.sync_copy(src_hbm.at[pl.ds(off, _CH)], sidx)
        pltpu.sync_copy(dst_hbm.at[pl.ds(off, _CH)], didx)
        pltpu.sync_copy(ew_hbm.at[pl.ds(off, _CH)], ew_v)
        pltpu.async_copy(u_hbm.at[sidx], rows, sem).wait()

        def scale(g, _):
            w16 = ew_v[pl.ds(g * 16, 16)]
            for l in range(16):
                w = w16[l]
                k = g * 16 + l
                for j in range(8):
                    rows[k, pl.ds(j * 16, 16)] = rows[k, pl.ds(j * 16, 16)] * w
            return 0
        lax.fori_loop(0, _CH // 16, scale, 0)
        pltpu.sync_copy(rows, acc_sh.at[didx], add=True)
        return 0
    lax.fori_loop(0, nch, body, 0)
    plsc.subcore_barrier()

    @pl.when(c == 0)
    def _():
        def wout(i, _):
            pltpu.sync_copy(acc_sh.at[pl.ds(s * sl + i * _CH, _CH)], rows)
            pltpu.sync_copy(rows, b0.at[pl.ds(s * sl + i * _CH, _CH)])
            return 0
        lax.fori_loop(0, nsub, wout, 0)

    @pl.when(c == 1)
    def _():
        def wout(i, _):
            pltpu.sync_copy(acc_sh.at[pl.ds(s * sl + i * _CH, _CH)], rows)
            pltpu.sync_copy(rows, b1.at[pl.ds(s * sl + i * _CH, _CH)])
            return 0
        lax.fori_loop(0, nsub, wout, 0)


def _acc2_call(u2, src, dst, ew):
    mesh = plsc.VectorSubcoreMesh(core_axis_name="c", subcore_axis_name="s")
    f = pl.kernel(
        _acc2_body,
        out_type=[jax.ShapeDtypeStruct((_NPAD, 128), jnp.float32)] * 2,
        mesh=mesh,
        scratch_types=[
            pltpu.VMEM((_CH,), jnp.int32),
            pltpu.VMEM((_CH,), jnp.int32),
            pltpu.VMEM((_CH,), jnp.float32),
            pltpu.VMEM((_CH, 128), jnp.float32),
            pltpu.VMEM((_CH, 128), jnp.float32),
            pltpu.VMEM_SHARED((_NPAD, 128), jnp.float32),
            pltpu.SemaphoreType.DMA,
        ],
    )
    return f(u2, src, dst, ew)


def _mlp_body(x_ref, sty_ref, dga_ref, dgb_ref,
              w1, b1, ws1, bs1, w2, b2, ws2, bs2,
              w3, b3, ws3, bs3, w4, b4, ws4, bs4,
              h_ref, u0_ref, u1_ref, u2_ref, u3_ref):
    h = x_ref[0]
    sty = sty_ref[0]

    def unit(h, W, b, Ws, bs):
        d = W.shape[1]
        y = jnp.dot(h, W, preferred_element_type=jnp.float32) + b[0]
        mu = jnp.mean(y, axis=0, keepdims=True)
        var = jnp.mean((y - mu) * (y - mu), axis=0, keepdims=True)
        yn = (y - mu) / jnp.sqrt(var + 1e-5)
        gb = jnp.dot(sty, Ws, preferred_element_type=jnp.float32) + bs[0]
        gamma = gb[:, :d]
        beta = gb[:, d:]
        return _leaky((1.0 + gamma) * yn + beta)

    h = unit(h, w1[...], b1[...], ws1[...], bs1[...])
    h = unit(h, w2[...], b2[...], ws2[...], bs2[...])
    h = unit(h, w3[...], b3[...], ws3[...], bs3[...])
    h = unit(h, w4[...], b4[...], ws4[...], bs4[...])
    h_ref[0] = h
    dis = _dis_of(dga_ref[0] + dgb_ref[0])
    u = dis * h
    u0_ref[0] = u[:, 0:128]
    u1_ref[0] = u[:, 128:256]
    u2_ref[0] = u[:, 256:384]
    u3_ref[0] = u[:, 384:512]


def _mlp_call(x, sty, dga, dgb, params):
    N, D = x.shape
    Bn = sty.shape[0]
    R = N // Bn
    full = lambda shp: pl.BlockSpec(shp, lambda i: (0,) * len(shp))
    blk3 = lambda r, d: pl.BlockSpec((1, r, d), lambda i: (i, 0, 0))
    w_args = []
    w_specs = []
    for li in range(1, 5):
        W = params['fc%d_W' % li]
        b = params['fc%d_b' % li].reshape(1, -1)
        Ws = params['fc%d_Ws' % li]
        bs = params['fc%d_bs' % li].reshape(1, -1)
        w_args += [W, b, Ws, bs]
        w_specs += [full(W.shape), full(b.shape), full(Ws.shape), full(bs.shape)]
    out_shape = [jax.ShapeDtypeStruct((Bn, R, 512), jnp.float32)] + [
        jax.ShapeDtypeStruct((Bn, R, 128), jnp.float32)] * 4
    in_specs = [blk3(R, D), pl.BlockSpec((1, 1, sty.shape[1]), lambda i: (i, 0, 0)),
                blk3(R, 1), blk3(R, 1)] + w_specs
    out_specs = [blk3(R, 512)] + [blk3(R, 128)] * 4
    outs = pl.pallas_call(
        _mlp_body, grid=(Bn,), in_specs=in_specs, out_specs=out_specs,
        out_shape=out_shape)(
            x.reshape(Bn, R, D), sty.reshape(Bn, 1, -1),
            dga.reshape(Bn, R, 1), dgb.reshape(Bn, R, 1), *w_args)
    return tuple(o.reshape(N, -1) for o in outs)


def _conv_mm_body(h_ref, acc_ref, dga_ref, dgb_ref, w0_ref, w1_ref, b_ref,
                  y_ref, st_ref):
    dis = _dis_of(dga_ref[0] + dgb_ref[0])
    ax = dis * acc_ref[0]
    y = (jnp.dot(h_ref[0], w0_ref[...], preferred_element_type=jnp.float32)
         - jnp.dot(ax, w1_ref[...], preferred_element_type=jnp.float32)
         + b_ref[0])
    y_ref[0] = y
    ps = jnp.sum(y, axis=0, keepdims=True)
    pq = jnp.sum(y * y, axis=0, keepdims=True)
    st_ref[...] = jnp.concatenate([ps[None], pq[None]], axis=1)


def _conv_mm_call(h, acc, dga, dgb, W0, W1, b):
    N, Din = h.shape
    Dout = W0.shape[1]
    R = 500
    G = N // R
    full = lambda shp: pl.BlockSpec(shp, lambda i: (0,) * len(shp))
    blk3 = lambda r, d: pl.BlockSpec((1, r, d), lambda i: (i, 0, 0))
    y, st = pl.pallas_call(
        _conv_mm_body, grid=(G,),
        in_specs=[blk3(R, Din), blk3(R, Din), blk3(R, 1), blk3(R, 1),
                  full(W0.shape), full(W1.shape), full((1, Dout))],
        out_specs=[blk3(R, Dout),
                   pl.BlockSpec((1, 2, Dout), lambda i: (i, 0, 0))],
        out_shape=[
            jax.ShapeDtypeStruct((G, R, Dout), jnp.float32),
            jax.ShapeDtypeStruct((G, 2, Dout), jnp.float32),
        ])(h.reshape(G, R, Din), acc.reshape(G, R, Din),
           dga.reshape(G, R, 1), dgb.reshape(G, R, 1), W0, W1, b.reshape(1, -1))
    return y.reshape(N, Dout), st


def _norm_body_with_next(y_ref, st_ref, dga_ref, dgb_ref, g_ref, bb_ref,
                         wn_ref, h1_ref, un_ref, *, n_rows):
    st = jnp.sum(st_ref[...], axis=0)
    mu = st[0:1] / n_rows
    msq = st[1:2] / n_rows
    var = msq - mu * mu
    z = (y_ref[0] - mu) / jnp.sqrt(var + 1e-5) * g_ref[0] + bb_ref[0]
    h1 = _leaky(z)
    h1_ref[0] = h1
    dis = _dis_of(dga_ref[0] + dgb_ref[0])
    un_ref[0] = dis * jnp.dot(h1, wn_ref[...], preferred_element_type=jnp.float32)


def _norm_next_call(y, st, dga, dgb, g, bb, Wn):
    N, Dout = y.shape
    R = 500
    G = N // R
    Dn = Wn.shape[1]
    full = lambda shp: pl.BlockSpec(shp, lambda i: (0,) * len(shp))
    blk3 = lambda r, d: pl.BlockSpec((1, r, d), lambda i: (i, 0, 0))
    body = functools.partial(_norm_body_with_next, n_rows=float(N))
    h1, un = pl.pallas_call(
        body, grid=(G,),
        in_specs=[blk3(R, Dout), full(st.shape), blk3(R, 1), blk3(R, 1),
                  full((1, Dout)), full((1, Dout)), full(Wn.shape)],
        out_specs=[blk3(R, Dout), blk3(R, Dn)],
        out_shape=[
            jax.ShapeDtypeStruct((G, R, Dout), jnp.float32),
            jax.ShapeDtypeStruct((G, R, Dn), jnp.float32),
        ])(y.reshape(G, R, Dout), st, dga.reshape(G, R, 1),
           dgb.reshape(G, R, 1), g.reshape(1, -1), bb.reshape(1, -1), Wn)
    return h1.reshape(N, Dout), un.reshape(N, Dn)


def _norm_body_final(y_ref, st_ref, g_ref, bb_ref, o_ref, *, n_rows):
    st = jnp.sum(st_ref[...], axis=0)
    mu = st[0:1] / n_rows
    msq = st[1:2] / n_rows
    var = msq - mu * mu
    z = (y_ref[0] - mu) / jnp.sqrt(var + 1e-5) * g_ref[0] + bb_ref[0]
    o_ref[0] = _leaky(z)


def _norm_final_call(y, st, g, bb):
    N, Dout = y.shape
    R = 500
    G = N // R
    full = lambda shp: pl.BlockSpec(shp, lambda i: (0,) * len(shp))
    blk3 = lambda r, d: pl.BlockSpec((1, r, d), lambda i: (i, 0, 0))
    body = functools.partial(_norm_body_final, n_rows=float(N))
    out = pl.pallas_call(
        body, grid=(G,),
        in_specs=[blk3(R, Dout), full(st.shape), full((1, Dout)), full((1, Dout))],
        out_specs=blk3(R, Dout),
        out_shape=jax.ShapeDtypeStruct((G, R, Dout), jnp.float32),
    )(y.reshape(G, R, Dout), st, g.reshape(1, -1), bb.reshape(1, -1))
    return out.reshape(N, Dout)


def _conv2_mm_body(h_ref, aa_ref, ab_ref, dga_ref, dgb_ref, w0_ref, b_ref,
                   y_ref, st_ref):
    dis = _dis_of(dga_ref[0] + dgb_ref[0])
    d = w0_ref.shape[1]
    ax = dis * (aa_ref[0][:, :d] + ab_ref[0][:, :d])
    y = (jnp.dot(h_ref[0], w0_ref[...], preferred_element_type=jnp.float32)
         - ax + b_ref[0])
    y_ref[0] = y
    ps = jnp.sum(y, axis=0, keepdims=True)
    pq = jnp.sum(y * y, axis=0, keepdims=True)
    st_ref[...] = jnp.concatenate([ps[None], pq[None]], axis=1)


def _conv2_mm_call(h1, acc2a, acc2b, dga, dgb, W0, b):
    N, Din = h1.shape
    Dout = W0.shape[1]
    R = 500
    G = N // R
    full = lambda shp: pl.BlockSpec(shp, lambda i: (0,) * len(shp))
    blk3 = lambda r, d: pl.BlockSpec((1, r, d), lambda i: (i, 0, 0))
    y, st = pl.pallas_call(
        _conv2_mm_body, grid=(G,),
        in_specs=[blk3(R, Din), blk3(R, 128), blk3(R, 128),
                  blk3(R, 1), blk3(R, 1), full(W0.shape), full((1, Dout))],
        out_specs=[blk3(R, Dout),
                   pl.BlockSpec((1, 2, Dout), lambda i: (i, 0, 0))],
        out_shape=[
            jax.ShapeDtypeStruct((G, R, Dout), jnp.float32),
            jax.ShapeDtypeStruct((G, 2, Dout), jnp.float32),
        ])(h1.reshape(G, R, Din), acc2a.reshape(G, R, 128),
           acc2b.reshape(G, R, 128), dga.reshape(G, R, 1),
           dgb.reshape(G, R, 1), W0, b.reshape(1, -1))
    return y.reshape(N, Dout), st


def kernel(x, style_vector, edge_index, edge_attr, batch_size, nroi, params):
    N, D = x.shape
    Bn = style_vector.shape[0]
    R = N // Bn

    src = edge_index[0].astype(jnp.int32)
    dst = edge_index[1].astype(jnp.int32)
    ew = edge_attr.astype(jnp.float32)

    dg0, dg1 = _deg_call(src, ew)
    dga = dg0[:N].reshape(N, 1)
    dgb = dg1[:N].reshape(N, 1)

    h, u0, u1, u2, u3 = _mlp_call(x, style_vector, dga, dgb, params)

    a0, a1, a2, a3 = _acc1_call((u0, u1, u2, u3), src, dst, ew)
    acc1 = jnp.concatenate(
        [a0[:N], a1[:N], a2[:N], a3[:N]], axis=1)

    y1, st1 = _conv_mm_call(h, acc1, dga, dgb,
                            params['conv1_W0'], params['conv1_W1'],
                            params['conv1_b'])
    w1n = jnp.pad(params['conv2_W1'], ((0, 0), (0, 64)))
    h1, u2w = _norm_next_call(y1, st1, dga, dgb,
                              params['conv1_g'], params['conv1_bb'], w1n)

    b0, b1 = _acc2_call(u2w, src, dst, ew)

    y2, st2 = _conv2_mm_call(h1, b0[:N], b1[:N], dga, dgb,
                             params['conv2_W0'], params['conv2_b'])
    out = _norm_final_call(y2, st2, params['conv2_g'], params['conv2_bb'])
    return out.reshape(Bn, R, out.shape[1])


# speedup vs baseline: 11.7079x; 2.1813x over previous
import functools

import jax
import jax.numpy as jnp
from jax import lax
from jax.experimental import pallas as pl
from jax.experimental.pallas import tpu as pltpu
from jax.experimental.pallas import tpu_sc as plsc

_NC = 2
_NS = 16
_NPAD = 10240
_CH = 80


def _leaky(v):
    return jnp.where(v > 0, v, 0.2 * v)


def _dis_of(deg):
    return jnp.where(deg > 0, 1.0 / jnp.sqrt(deg + 1e-12), 0.0)


def _deg_body(src_hbm, ew_hbm, deg0_hbm, deg1_hbm, sidx, ewv, z_v, acc_sh, sem):
    c = lax.axis_index("c")
    s = lax.axis_index("s")
    nrow = sidx.shape[0]
    sl = _NPAD // _NS

    def zb(i, _):
        z_v[pl.ds(i * 16, 16)] = jnp.zeros((16,), jnp.float32)
        return 0
    lax.fori_loop(0, sl // 16, zb, 0)
    pltpu.sync_copy(z_v, acc_sh.at[pl.ds(s * sl, sl)])

    wid = c * _NS + s
    pltpu.sync_copy(src_hbm.at[wid], sidx)
    pltpu.sync_copy(ew_hbm.at[wid], ewv)
    plsc.subcore_barrier()

    def body(i, _):
        pltpu.sync_copy(ewv.at[i], acc_sh.at[sidx.at[i]], add=True)
        return 0
    lax.fori_loop(0, nrow, body, 0)
    plsc.subcore_barrier()

    @pl.when(c == 0)
    def _():
        pltpu.sync_copy(acc_sh.at[pl.ds(s * sl, sl)], deg0_hbm.at[pl.ds(s * sl, sl)])

    @pl.when(c == 1)
    def _():
        pltpu.sync_copy(acc_sh.at[pl.ds(s * sl, sl)], deg1_hbm.at[pl.ds(s * sl, sl)])


def _deg_call(src2d, ew2d):
    nrow = src2d.shape[1]
    mesh = plsc.VectorSubcoreMesh(core_axis_name="c", subcore_axis_name="s")
    f = pl.kernel(
        _deg_body,
        out_type=[jax.ShapeDtypeStruct((_NPAD,), jnp.float32),
                  jax.ShapeDtypeStruct((_NPAD,), jnp.float32)],
        mesh=mesh,
        scratch_types=[
            pltpu.VMEM((nrow, _CH), jnp.int32),
            pltpu.VMEM((nrow, _CH), jnp.float32),
            pltpu.VMEM((_NPAD // _NS,), jnp.float32),
            pltpu.VMEM_SHARED((_NPAD,), jnp.float32),
            pltpu.SemaphoreType.DMA,
        ],
    )
    return f(src2d, ew2d)


_GR = 25


def _zero_rows(buf, nj):
    def zb(k, _):
        for j in range(nj):
            buf[k, pl.ds(j * 16, 16)] = jnp.zeros((16,), jnp.float32)
        return 0
    lax.fori_loop(0, _CH, zb, 0)


def _zero_acc(acc_sh, zrows, s):
    sl = _NPAD // _NS

    def z(i, _):
        pltpu.sync_copy(zrows, acc_sh.at[pl.ds(s * sl + i * _CH, _CH)])
        return 0
    lax.fori_loop(0, sl // _CH, z, 0)


def _writeout(acc_sh, out_hbm, buf, s):
    sl = _NPAD // _NS

    def w(i, _):
        pltpu.sync_copy(acc_sh.at[pl.ds(s * sl + i * _CH, _CH)], buf)
        pltpu.sync_copy(buf, out_hbm.at[pl.ds(s * sl + i * _CH, _CH)])
        return 0
    lax.fori_loop(0, sl // _CH, w, 0)


def _edge_sweeps(u_hbm, acc_sh, src4, dst4, ew4, tid,
                 sidx, didx, ewv, r0, r1, g0, g1, s0, s1, nj):
    nswp = src4.shape[1]
    npair = _GR // 2

    def scale(rows, wi):
        def grp(g, _):
            w16 = ewv[wi, pl.ds(g * 16, 16)]
            for l in range(16):
                w = w16[l]
                k = g * 16 + l
                for j in range(nj):
                    rows[k, pl.ds(j * 16, 16)] = rows[k, pl.ds(j * 16, 16)] * w
            return 0
        lax.fori_loop(0, _CH // 16, grp, 0)

    def sweep(g, _):
        pltpu.sync_copy(src4.at[tid, g], sidx)
        pltpu.sync_copy(dst4.at[tid, g], didx)
        pltpu.sync_copy(ew4.at[tid, g], ewv)
        pltpu.async_copy(u_hbm.at[sidx.at[0]], r0, g0)
        pltpu.async_copy(u_hbm.at[sidx.at[1]], r1, g1)

        def body(i, _):
            i0 = 2 * i
            i1p = 2 * i + 1
            pltpu.make_async_copy(u_hbm.at[sidx.at[0]], r0, g0).wait()
            scale(r0, i0)
            pltpu.async_copy(r0, acc_sh.at[didx.at[i0]], s0, add=True)
            pltpu.make_async_copy(u_hbm.at[sidx.at[1]], r1, g1).wait()
            scale(r1, i1p)
            pltpu.async_copy(r1, acc_sh.at[didx.at[i1p]], s1, add=True)

            @pl.when(i0 + 2 <= _GR - 1)
            def _():
                pltpu.make_async_copy(r0, acc_sh.at[didx.at[i0]], s0).wait()
                pltpu.async_copy(u_hbm.at[sidx.at[i0 + 2]], r0, g0)

            @pl.when(i1p + 2 <= _GR - 1)
            def _():
                pltpu.make_async_copy(r1, acc_sh.at[didx.at[i1p]], s1).wait()
                pltpu.async_copy(u_hbm.at[sidx.at[i1p + 2]], r1, g1)
            return 0
        lax.fori_loop(0, npair, body, 0)
        pltpu.make_async_copy(r1, acc_sh.at[didx.at[_GR - 2]], s1).wait()
        pltpu.make_async_copy(u_hbm.at[sidx.at[_GR - 1]], r0, g0).wait()
        scale(r0, _GR - 1)
        pltpu.async_copy(r0, acc_sh.at[didx.at[_GR - 1]], s0, add=True)
        pltpu.make_async_copy(r0, acc_sh.at[didx.at[_GR - 1]], s0).wait()
        return 0
    lax.fori_loop(0, nswp, sweep, 0)


def _acc1_body(u0, u1, u2, u3, src4, dst4, ew4,
               a0, a1, a2, a3,
               sidx, didx, ewv, r0, r1, acc_sh, g0, g1, s0, s1):
    c = lax.axis_index("c")
    s = lax.axis_index("s")
    u_all = (u0, u1, u2, u3)
    out_all = (a0, a1, a2, a3)
    for p in range(4):
        @pl.when(c == p // 2)
        def _(p=p):
            _zero_rows(r0, 8)
            _zero_acc(acc_sh, r0, s)
            plsc.subcore_barrier()
            _edge_sweeps(u_all[p], acc_sh, src4, dst4, ew4, s,
                         sidx, didx, ewv, r0, r1, g0, g1, s0, s1, 8)
            plsc.subcore_barrier()
            _writeout(acc_sh, out_all[p], r0, s)
            plsc.subcore_barrier()


def _acc1_call(u_panels, src4, dst4, ew4):
    mesh = plsc.VectorSubcoreMesh(core_axis_name="c", subcore_axis_name="s")
    f = pl.kernel(
        _acc1_body,
        out_type=[jax.ShapeDtypeStruct((_NPAD, 128), jnp.float32)] * 4,
        mesh=mesh,
        scratch_types=[
            pltpu.VMEM((_GR, _CH), jnp.int32),
            pltpu.VMEM((_GR, _CH), jnp.int32),
            pltpu.VMEM((_GR, _CH), jnp.float32),
            pltpu.VMEM((_CH, 128), jnp.float32),
            pltpu.VMEM((_CH, 128), jnp.float32),
            pltpu.VMEM_SHARED((_NPAD, 128), jnp.float32),
            pltpu.SemaphoreType.DMA,
            pltpu.SemaphoreType.DMA,
            pltpu.SemaphoreType.DMA,
            pltpu.SemaphoreType.DMA,
        ],
    )
    return f(*u_panels, src4, dst4, ew4)


def _acc2_body(u_hbm, src4, dst4, ew4, b0, b1,
               sidx, didx, ewv, r0, r1, acc_sh, g0, g1, s0, s1):
    c = lax.axis_index("c")
    s = lax.axis_index("s")
    wid = c * _NS + s
    _zero_rows(r0, 8)
    _zero_acc(acc_sh, r0, s)
    plsc.subcore_barrier()
    _edge_sweeps(u_hbm, acc_sh, src4, dst4, ew4, wid,
                 sidx, didx, ewv, r0, r1, g0, g1, s0, s1, 8)
    plsc.subcore_barrier()

    @pl.when(c == 0)
    def _():
        _writeout(acc_sh, b0, r0, s)

    @pl.when(c == 1)
    def _():
        _writeout(acc_sh, b1, r0, s)


def _acc2_call(u2, src4, dst4, ew4):
    mesh = plsc.VectorSubcoreMesh(core_axis_name="c", subcore_axis_name="s")
    f = pl.kernel(
        _acc2_body,
        out_type=[jax.ShapeDtypeStruct((_NPAD, 128), jnp.float32)] * 2,
        mesh=mesh,
        scratch_types=[
            pltpu.VMEM((_GR, _CH), jnp.int32),
            pltpu.VMEM((_GR, _CH), jnp.int32),
            pltpu.VMEM((_GR, _CH), jnp.float32),
            pltpu.VMEM((_CH, 128), jnp.float32),
            pltpu.VMEM((_CH, 128), jnp.float32),
            pltpu.VMEM_SHARED((_NPAD, 128), jnp.float32),
            pltpu.SemaphoreType.DMA,
            pltpu.SemaphoreType.DMA,
            pltpu.SemaphoreType.DMA,
            pltpu.SemaphoreType.DMA,
        ],
    )
    return f(u2, src4, dst4, ew4)


def _mlp_body(x_ref, sty_ref, dga_ref, dgb_ref,
              w1, b1, ws1, bs1, w2, b2, ws2, bs2,
              w3, b3, ws3, bs3, w4, b4, ws4, bs4,
              h_ref, u0_ref, u1_ref, u2_ref, u3_ref):
    h = x_ref[0]
    sty = sty_ref[0]

    def unit(h, W, b, Ws, bs):
        d = W.shape[1]
        y = jnp.dot(h, W, preferred_element_type=jnp.float32) + b[0]
        mu = jnp.mean(y, axis=0, keepdims=True)
        var = jnp.mean((y - mu) * (y - mu), axis=0, keepdims=True)
        yn = (y - mu) / jnp.sqrt(var + 1e-5)
        gb = jnp.dot(sty, Ws, preferred_element_type=jnp.float32) + bs[0]
        gamma = gb[:, :d]
        beta = gb[:, d:]
        return _leaky((1.0 + gamma) * yn + beta)

    h = unit(h, w1[...], b1[...], ws1[...], bs1[...])
    h = unit(h, w2[...], b2[...], ws2[...], bs2[...])
    h = unit(h, w3[...], b3[...], ws3[...], bs3[...])
    h = unit(h, w4[...], b4[...], ws4[...], bs4[...])
    h_ref[0] = h
    dis = _dis_of(dga_ref[0] + dgb_ref[0])
    u = dis * h
    u0_ref[0] = u[:, 0:128]
    u1_ref[0] = u[:, 128:256]
    u2_ref[0] = u[:, 256:384]
    u3_ref[0] = u[:, 384:512]


def _mlp_call(x, sty, dga, dgb, params):
    N, D = x.shape
    Bn = sty.shape[0]
    R = N // Bn
    full = lambda shp: pl.BlockSpec(shp, lambda i: (0,) * len(shp))
    blk3 = lambda r, d: pl.BlockSpec((1, r, d), lambda i: (i, 0, 0))
    w_args = []
    w_specs = []
    for li in range(1, 5):
        W = params['fc%d_W' % li]
        b = params['fc%d_b' % li].reshape(1, -1)
        Ws = params['fc%d_Ws' % li]
        bs = params['fc%d_bs' % li].reshape(1, -1)
        w_args += [W, b, Ws, bs]
        w_specs += [full(W.shape), full(b.shape), full(Ws.shape), full(bs.shape)]
    out_shape = [jax.ShapeDtypeStruct((Bn, R, 512), jnp.float32)] + [
        jax.ShapeDtypeStruct((Bn, R, 128), jnp.float32)] * 4
    in_specs = [blk3(R, D), pl.BlockSpec((1, 1, sty.shape[1]), lambda i: (i, 0, 0)),
                blk3(R, 1), blk3(R, 1)] + w_specs
    out_specs = [blk3(R, 512)] + [blk3(R, 128)] * 4
    outs = pl.pallas_call(
        _mlp_body, grid=(Bn,), in_specs=in_specs, out_specs=out_specs,
        out_shape=out_shape)(
            x.reshape(Bn, R, D), sty.reshape(Bn, 1, -1),
            dga.reshape(Bn, R, 1), dgb.reshape(Bn, R, 1), *w_args)
    return tuple(o.reshape(N, -1) for o in outs)


def _conv_mm_body(h_ref, acc_ref, dga_ref, dgb_ref, w0_ref, w1_ref, b_ref,
                  y_ref, st_ref):
    dis = _dis_of(dga_ref[0] + dgb_ref[0])
    ax = dis * acc_ref[0]
    y = (jnp.dot(h_ref[0], w0_ref[...], preferred_element_type=jnp.float32)
         - jnp.dot(ax, w1_ref[...], preferred_element_type=jnp.float32)
         + b_ref[0])
    y_ref[0] = y
    ps = jnp.sum(y, axis=0, keepdims=True)
    pq = jnp.sum(y * y, axis=0, keepdims=True)
    st_ref[...] = jnp.concatenate([ps[None], pq[None]], axis=1)


def _conv_mm_call(h, acc, dga, dgb, W0, W1, b):
    N, Din = h.shape
    Dout = W0.shape[1]
    R = 500
    G = N // R
    full = lambda shp: pl.BlockSpec(shp, lambda i: (0,) * len(shp))
    blk3 = lambda r, d: pl.BlockSpec((1, r, d), lambda i: (i, 0, 0))
    y, st = pl.pallas_call(
        _conv_mm_body, grid=(G,),
        in_specs=[blk3(R, Din), blk3(R, Din), blk3(R, 1), blk3(R, 1),
                  full(W0.shape), full(W1.shape), full((1, Dout))],
        out_specs=[blk3(R, Dout),
                   pl.BlockSpec((1, 2, Dout), lambda i: (i, 0, 0))],
        out_shape=[
            jax.ShapeDtypeStruct((G, R, Dout), jnp.float32),
            jax.ShapeDtypeStruct((G, 2, Dout), jnp.float32),
        ])(h.reshape(G, R, Din), acc.reshape(G, R, Din),
           dga.reshape(G, R, 1), dgb.reshape(G, R, 1), W0, W1, b.reshape(1, -1))
    return y.reshape(N, Dout), st


def _norm_body_with_next(y_ref, st_ref, dga_ref, dgb_ref, g_ref, bb_ref,
                         wn_ref, h1_ref, un_ref, *, n_rows):
    st = jnp.sum(st_ref[...], axis=0)
    mu = st[0:1] / n_rows
    msq = st[1:2] / n_rows
    var = msq - mu * mu
    z = (y_ref[0] - mu) / jnp.sqrt(var + 1e-5) * g_ref[0] + bb_ref[0]
    h1 = _leaky(z)
    h1_ref[0] = h1
    dis = _dis_of(dga_ref[0] + dgb_ref[0])
    un_ref[0] = dis * jnp.dot(h1, wn_ref[...], preferred_element_type=jnp.float32)


def _norm_next_call(y, st, dga, dgb, g, bb, Wn):
    N, Dout = y.shape
    R = 500
    G = N // R
    Dn = Wn.shape[1]
    full = lambda shp: pl.BlockSpec(shp, lambda i: (0,) * len(shp))
    blk3 = lambda r, d: pl.BlockSpec((1, r, d), lambda i: (i, 0, 0))
    body = functools.partial(_norm_body_with_next, n_rows=float(N))
    h1, un = pl.pallas_call(
        body, grid=(G,),
        in_specs=[blk3(R, Dout), full(st.shape), blk3(R, 1), blk3(R, 1),
                  full((1, Dout)), full((1, Dout)), full(Wn.shape)],
        out_specs=[blk3(R, Dout), blk3(R, Dn)],
        out_shape=[
            jax.ShapeDtypeStruct((G, R, Dout), jnp.float32),
            jax.ShapeDtypeStruct((G, R, Dn), jnp.float32),
        ])(y.reshape(G, R, Dout), st, dga.reshape(G, R, 1),
           dgb.reshape(G, R, 1), g.reshape(1, -1), bb.reshape(1, -1), Wn)
    return h1.reshape(N, Dout), un.reshape(N, Dn)


def _norm_body_final(y_ref, st_ref, g_ref, bb_ref, o_ref, *, n_rows):
    st = jnp.sum(st_ref[...], axis=0)
    mu = st[0:1] / n_rows
    msq = st[1:2] / n_rows
    var = msq - mu * mu
    z = (y_ref[0] - mu) / jnp.sqrt(var + 1e-5) * g_ref[0] + bb_ref[0]
    o_ref[0] = _leaky(z)


def _norm_final_call(y, st, g, bb):
    N, Dout = y.shape
    R = 500
    G = N // R
    full = lambda shp: pl.BlockSpec(shp, lambda i: (0,) * len(shp))
    blk3 = lambda r, d: pl.BlockSpec((1, r, d), lambda i: (i, 0, 0))
    body = functools.partial(_norm_body_final, n_rows=float(N))
    out = pl.pallas_call(
        body, grid=(G,),
        in_specs=[blk3(R, Dout), full(st.shape), full((1, Dout)), full((1, Dout))],
        out_specs=blk3(R, Dout),
        out_shape=jax.ShapeDtypeStruct((G, R, Dout), jnp.float32),
    )(y.reshape(G, R, Dout), st, g.reshape(1, -1), bb.reshape(1, -1))
    return out.reshape(N, Dout)


def _conv2_mm_body(h_ref, aa_ref, ab_ref, dga_ref, dgb_ref, w0_ref, b_ref,
                   y_ref, st_ref):
    dis = _dis_of(dga_ref[0] + dgb_ref[0])
    d = w0_ref.shape[1]
    ax = dis * (aa_ref[0][:, :d] + ab_ref[0][:, :d])
    y = (jnp.dot(h_ref[0], w0_ref[...], preferred_element_type=jnp.float32)
         - ax + b_ref[0])
    y_ref[0] = y
    ps = jnp.sum(y, axis=0, keepdims=True)
    pq = jnp.sum(y * y, axis=0, keepdims=True)
    st_ref[...] = jnp.concatenate([ps[None], pq[None]], axis=1)


def _conv2_mm_call(h1, acc2a, acc2b, dga, dgb, W0, b):
    N, Din = h1.shape
    Dout = W0.shape[1]
    R = 500
    G = N // R
    full = lambda shp: pl.BlockSpec(shp, lambda i: (0,) * len(shp))
    blk3 = lambda r, d: pl.BlockSpec((1, r, d), lambda i: (i, 0, 0))
    y, st = pl.pallas_call(
        _conv2_mm_body, grid=(G,),
        in_specs=[blk3(R, Din), blk3(R, 128), blk3(R, 128),
                  blk3(R, 1), blk3(R, 1), full(W0.shape), full((1, Dout))],
        out_specs=[blk3(R, Dout),
                   pl.BlockSpec((1, 2, Dout), lambda i: (i, 0, 0))],
        out_shape=[
            jax.ShapeDtypeStruct((G, R, Dout), jnp.float32),
            jax.ShapeDtypeStruct((G, 2, Dout), jnp.float32),
        ])(h1.reshape(G, R, Din), acc2a.reshape(G, R, 128),
           acc2b.reshape(G, R, 128), dga.reshape(G, R, 1),
           dgb.reshape(G, R, 1), W0, b.reshape(1, -1))
    return y.reshape(N, Dout), st


def kernel(x, style_vector, edge_index, edge_attr, batch_size, nroi, params):
    N, D = x.shape
    Bn = style_vector.shape[0]
    R = N // Bn

    E = edge_index.shape[1]
    src = edge_index[0].astype(jnp.int32)
    dst = edge_index[1].astype(jnp.int32)
    ew = edge_attr.astype(jnp.float32)
    nw = _NC * _NS
    gsz = _GR * _CH
    src32 = src.reshape(nw, E // (nw * gsz), _GR, _CH)
    dst32 = dst.reshape(nw, E // (nw * gsz), _GR, _CH)
    ew32 = ew.reshape(nw, E // (nw * gsz), _GR, _CH)
    src16 = src.reshape(_NS, E // (_NS * gsz), _GR, _CH)
    dst16 = dst.reshape(_NS, E // (_NS * gsz), _GR, _CH)
    ew16 = ew.reshape(_NS, E // (_NS * gsz), _GR, _CH)
    srcd = src.reshape(nw, E // (nw * _CH), _CH)
    ewd = ew.reshape(nw, E // (nw * _CH), _CH)

    dg0, dg1 = _deg_call(srcd, ewd)
    dga = dg0[:N].reshape(N, 1)
    dgb = dg1[:N].reshape(N, 1)

    h, u0, u1, u2, u3 = _mlp_call(x, style_vector, dga, dgb, params)

    a0, a1, a2, a3 = _acc1_call((u0, u1, u2, u3), src16, dst16, ew16)
    acc1 = jnp.concatenate(
        [a0[:N], a1[:N], a2[:N], a3[:N]], axis=1)

    y1, st1 = _conv_mm_call(h, acc1, dga, dgb,
                            params['conv1_W0'], params['conv1_W1'],
                            params['conv1_b'])
    w1n = jnp.pad(params['conv2_W1'], ((0, 0), (0, 64)))
    h1, u2w = _norm_next_call(y1, st1, dga, dgb,
                              params['conv1_g'], params['conv1_bb'], w1n)

    b0, b1 = _acc2_call(u2w, src32, dst32, ew32)

    y2, st2 = _conv2_mm_call(h1, b0[:N], b1[:N], dga, dgb,
                             params['conv2_W0'], params['conv2_b'])
    out = _norm_final_call(y2, st2, params['conv2_g'], params['conv2_bb'])
    return out.reshape(Bn, R, out.shape[1])


# speedup vs baseline: 13.5399x; 1.1565x over previous
import functools

import jax
import jax.numpy as jnp
from jax import lax
from jax.experimental import pallas as pl
from jax.experimental.pallas import tpu as pltpu
from jax.experimental.pallas import tpu_sc as plsc

_NC = 2
_NS = 16
_NPAD = 10240
_CH = 80


def _leaky(v):
    return jnp.where(v > 0, v, 0.2 * v)


def _dis_of(deg):
    return jnp.where(deg > 0, 1.0 / jnp.sqrt(deg + 1e-12), 0.0)


def _deg_body(src_hbm, ew_hbm, deg0_hbm, deg1_hbm, sidx, ewv, z_v, acc_sh, sem):
    c = lax.axis_index("c")
    s = lax.axis_index("s")
    nrow = sidx.shape[0]
    sl = _NPAD // _NS

    def zb(i, _):
        z_v[pl.ds(i * 16, 16)] = jnp.zeros((16,), jnp.float32)
        return 0
    lax.fori_loop(0, sl // 16, zb, 0)
    pltpu.sync_copy(z_v, acc_sh.at[pl.ds(s * sl, sl)])

    wid = c * _NS + s
    pltpu.sync_copy(src_hbm.at[wid], sidx)
    pltpu.sync_copy(ew_hbm.at[wid], ewv)
    plsc.subcore_barrier()

    def body(i, _):
        pltpu.sync_copy(ewv.at[i], acc_sh.at[sidx.at[i]], add=True)
        return 0
    lax.fori_loop(0, nrow, body, 0)
    plsc.subcore_barrier()

    @pl.when(c == 0)
    def _():
        pltpu.sync_copy(acc_sh.at[pl.ds(s * sl, sl)], deg0_hbm.at[pl.ds(s * sl, sl)])

    @pl.when(c == 1)
    def _():
        pltpu.sync_copy(acc_sh.at[pl.ds(s * sl, sl)], deg1_hbm.at[pl.ds(s * sl, sl)])


def _deg_call(src2d, ew2d):
    nrow = src2d.shape[1]
    mesh = plsc.VectorSubcoreMesh(core_axis_name="c", subcore_axis_name="s")
    f = pl.kernel(
        _deg_body,
        out_type=[jax.ShapeDtypeStruct((_NPAD,), jnp.float32),
                  jax.ShapeDtypeStruct((_NPAD,), jnp.float32)],
        mesh=mesh,
        scratch_types=[
            pltpu.VMEM((nrow, _CH), jnp.int32),
            pltpu.VMEM((nrow, _CH), jnp.float32),
            pltpu.VMEM((_NPAD // _NS,), jnp.float32),
            pltpu.VMEM_SHARED((_NPAD,), jnp.float32),
            pltpu.SemaphoreType.DMA,
        ],
    )
    return f(src2d, ew2d)


_GR = 25


def _zero_rows(buf, nj):
    def zb(k, _):
        for j in range(nj):
            buf[k, pl.ds(j * 16, 16)] = jnp.zeros((16,), jnp.float32)
        return 0
    lax.fori_loop(0, _CH, zb, 0)


def _zero_acc(acc_sh, zrows, s):
    sl = _NPAD // _NS

    def z(i, _):
        pltpu.sync_copy(zrows, acc_sh.at[pl.ds(s * sl + i * _CH, _CH)])
        return 0
    lax.fori_loop(0, sl // _CH, z, 0)


def _writeout(acc_sh, out_hbm, buf, s):
    sl = _NPAD // _NS

    def w(i, _):
        pltpu.sync_copy(acc_sh.at[pl.ds(s * sl + i * _CH, _CH)], buf)
        pltpu.sync_copy(buf, out_hbm.at[pl.ds(s * sl + i * _CH, _CH)])
        return 0
    lax.fori_loop(0, sl // _CH, w, 0)


def _edge_sweeps(u_hbm, acc_sh, src4, dst4, ew4, tid,
                 sidx, didx, ewv, rbufs, gsems, ssems, nj):
    nswp = src4.shape[1]
    ntri = _GR // 3

    def scale(rows, wi):
        def grp(g, _):
            w16 = ewv[wi, pl.ds(g * 16, 16)]
            for l in range(16):
                w = w16[l]
                k = g * 16 + l
                for j in range(nj):
                    rows[k, pl.ds(j * 16, 16)] = rows[k, pl.ds(j * 16, 16)] * w
            return 0
        lax.fori_loop(0, _CH // 16, grp, 0)

    r0, r1, r2 = rbufs
    g0, g1, g2 = gsems
    s0, s1, s2 = ssems

    def sweep(g, _):
        pltpu.sync_copy(src4.at[tid, g], sidx)
        pltpu.sync_copy(dst4.at[tid, g], didx)
        pltpu.sync_copy(ew4.at[tid, g], ewv)
        for b in range(3):
            pltpu.async_copy(u_hbm.at[sidx.at[b]], rbufs[b], gsems[b])

        def body(i, _):
            for b in range(3):
                k = 3 * i + b
                pltpu.make_async_copy(u_hbm.at[sidx.at[0]], rbufs[b], gsems[b]).wait()
                scale(rbufs[b], k)
                pltpu.async_copy(rbufs[b], acc_sh.at[didx.at[k]], ssems[b], add=True)

                @pl.when(k + 3 <= _GR - 1)
                def _(b=b, k=k):
                    pltpu.make_async_copy(rbufs[b], acc_sh.at[didx.at[k]], ssems[b]).wait()
                    pltpu.async_copy(u_hbm.at[sidx.at[k + 3]], rbufs[b], gsems[b])
            return 0
        lax.fori_loop(0, ntri, body, 0)
        pltpu.make_async_copy(u_hbm.at[sidx.at[0]], r0, g0).wait()
        scale(r0, _GR - 1)
        pltpu.async_copy(r0, acc_sh.at[didx.at[_GR - 1]], s0, add=True)
        pltpu.make_async_copy(r0, acc_sh.at[didx.at[_GR - 1]], s0).wait()
        pltpu.make_async_copy(r1, acc_sh.at[didx.at[_GR - 3]], s1).wait()
        pltpu.make_async_copy(r2, acc_sh.at[didx.at[_GR - 2]], s2).wait()
        return 0
    lax.fori_loop(0, nswp, sweep, 0)


def _acc1_body(u0, u1, u2, u3, src4, dst4, ew4,
               a0, a1, a2, a3,
               sidx, didx, ewv, r0, r1, r2, acc_sh,
               g0, g1, g2, s0, s1, s2):
    c = lax.axis_index("c")
    s = lax.axis_index("s")
    u_all = (u0, u1, u2, u3)
    out_all = (a0, a1, a2, a3)
    for p in range(4):
        @pl.when(c == p // 2)
        def _(p=p):
            _zero_rows(r0, 8)
            _zero_acc(acc_sh, r0, s)
            plsc.subcore_barrier()
            _edge_sweeps(u_all[p], acc_sh, src4, dst4, ew4, s,
                         sidx, didx, ewv, (r0, r1, r2),
                         (g0, g1, g2), (s0, s1, s2), 8)
            plsc.subcore_barrier()
            _writeout(acc_sh, out_all[p], r0, s)
            plsc.subcore_barrier()


def _acc1_call(u_panels, src4, dst4, ew4):
    mesh = plsc.VectorSubcoreMesh(core_axis_name="c", subcore_axis_name="s")
    f = pl.kernel(
        _acc1_body,
        out_type=[jax.ShapeDtypeStruct((_NPAD, 128), jnp.float32)] * 4,
        mesh=mesh,
        scratch_types=[
            pltpu.VMEM((_GR, _CH), jnp.int32),
            pltpu.VMEM((_GR, _CH), jnp.int32),
            pltpu.VMEM((_GR, _CH), jnp.float32),
            pltpu.VMEM((_CH, 128), jnp.float32),
            pltpu.VMEM((_CH, 128), jnp.float32),
            pltpu.VMEM((_CH, 128), jnp.float32),
            pltpu.VMEM_SHARED((_NPAD, 128), jnp.float32),
            pltpu.SemaphoreType.DMA,
            pltpu.SemaphoreType.DMA,
            pltpu.SemaphoreType.DMA,
            pltpu.SemaphoreType.DMA,
            pltpu.SemaphoreType.DMA,
            pltpu.SemaphoreType.DMA,
        ],
    )
    return f(*u_panels, src4, dst4, ew4)


def _acc2_body(u_hbm, src4, dst4, ew4, b0, b1,
               sidx, didx, ewv, r0, r1, r2, acc_sh,
               g0, g1, g2, s0, s1, s2):
    c = lax.axis_index("c")
    s = lax.axis_index("s")
    wid = c * _NS + s
    _zero_rows(r0, 8)
    _zero_acc(acc_sh, r0, s)
    plsc.subcore_barrier()
    _edge_sweeps(u_hbm, acc_sh, src4, dst4, ew4, wid,
                 sidx, didx, ewv, (r0, r1, r2),
                 (g0, g1, g2), (s0, s1, s2), 8)
    plsc.subcore_barrier()

    @pl.when(c == 0)
    def _():
        _writeout(acc_sh, b0, r0, s)

    @pl.when(c == 1)
    def _():
        _writeout(acc_sh, b1, r0, s)


def _acc2_call(u2, src4, dst4, ew4):
    mesh = plsc.VectorSubcoreMesh(core_axis_name="c", subcore_axis_name="s")
    f = pl.kernel(
        _acc2_body,
        out_type=[jax.ShapeDtypeStruct((_NPAD, 128), jnp.float32)] * 2,
        mesh=mesh,
        scratch_types=[
            pltpu.VMEM((_GR, _CH), jnp.int32),
            pltpu.VMEM((_GR, _CH), jnp.int32),
            pltpu.VMEM((_GR, _CH), jnp.float32),
            pltpu.VMEM((_CH, 128), jnp.float32),
            pltpu.VMEM((_CH, 128), jnp.float32),
            pltpu.VMEM((_CH, 128), jnp.float32),
            pltpu.VMEM_SHARED((_NPAD, 128), jnp.float32),
            pltpu.SemaphoreType.DMA,
            pltpu.SemaphoreType.DMA,
            pltpu.SemaphoreType.DMA,
            pltpu.SemaphoreType.DMA,
            pltpu.SemaphoreType.DMA,
            pltpu.SemaphoreType.DMA,
        ],
    )
    return f(u2, src4, dst4, ew4)


def _mlp_body(x_ref, sty_ref, dga_ref, dgb_ref,
              w1, b1, ws1, bs1, w2, b2, ws2, bs2,
              w3, b3, ws3, bs3, w4, b4, ws4, bs4,
              h_ref, u0_ref, u1_ref, u2_ref, u3_ref):
    h = x_ref[0]
    sty = sty_ref[0]

    def unit(h, W, b, Ws, bs):
        d = W.shape[1]
        y = jnp.dot(h, W, preferred_element_type=jnp.float32) + b[0]
        mu = jnp.mean(y, axis=0, keepdims=True)
        var = jnp.mean((y - mu) * (y - mu), axis=0, keepdims=True)
        yn = (y - mu) / jnp.sqrt(var + 1e-5)
        gb = jnp.dot(sty, Ws, preferred_element_type=jnp.float32) + bs[0]
        gamma = gb[:, :d]
        beta = gb[:, d:]
        return _leaky((1.0 + gamma) * yn + beta)

    h = unit(h, w1[...], b1[...], ws1[...], bs1[...])
    h = unit(h, w2[...], b2[...], ws2[...], bs2[...])
    h = unit(h, w3[...], b3[...], ws3[...], bs3[...])
    h = unit(h, w4[...], b4[...], ws4[...], bs4[...])
    h_ref[0] = h
    dis = _dis_of(dga_ref[0] + dgb_ref[0])
    u = dis * h
    u0_ref[0] = u[:, 0:128]
    u1_ref[0] = u[:, 128:256]
    u2_ref[0] = u[:, 256:384]
    u3_ref[0] = u[:, 384:512]


def _mlp_call(x, sty, dga, dgb, params):
    N, D = x.shape
    Bn = sty.shape[0]
    R = N // Bn
    full = lambda shp: pl.BlockSpec(shp, lambda i: (0,) * len(shp))
    blk3 = lambda r, d: pl.BlockSpec((1, r, d), lambda i: (i, 0, 0))
    w_args = []
    w_specs = []
    for li in range(1, 5):
        W = params['fc%d_W' % li]
        b = params['fc%d_b' % li].reshape(1, -1)
        Ws = params['fc%d_Ws' % li]
        bs = params['fc%d_bs' % li].reshape(1, -1)
        w_args += [W, b, Ws, bs]
        w_specs += [full(W.shape), full(b.shape), full(Ws.shape), full(bs.shape)]
    out_shape = [jax.ShapeDtypeStruct((Bn, R, 512), jnp.float32)] + [
        jax.ShapeDtypeStruct((Bn, R, 128), jnp.float32)] * 4
    in_specs = [blk3(R, D), pl.BlockSpec((1, 1, sty.shape[1]), lambda i: (i, 0, 0)),
                blk3(R, 1), blk3(R, 1)] + w_specs
    out_specs = [blk3(R, 512)] + [blk3(R, 128)] * 4
    outs = pl.pallas_call(
        _mlp_body, grid=(Bn,), in_specs=in_specs, out_specs=out_specs,
        out_shape=out_shape)(
            x.reshape(Bn, R, D), sty.reshape(Bn, 1, -1),
            dga.reshape(Bn, R, 1), dgb.reshape(Bn, R, 1), *w_args)
    return tuple(o.reshape(N, -1) for o in outs)


def _conv_mm_body(h_ref, acc_ref, dga_ref, dgb_ref, w0_ref, w1_ref, b_ref,
                  y_ref, st_ref):
    dis = _dis_of(dga_ref[0] + dgb_ref[0])
    ax = dis * acc_ref[0]
    y = (jnp.dot(h_ref[0], w0_ref[...], preferred_element_type=jnp.float32)
         - jnp.dot(ax, w1_ref[...], preferred_element_type=jnp.float32)
         + b_ref[0])
    y_ref[0] = y
    ps = jnp.sum(y, axis=0, keepdims=True)
    pq = jnp.sum(y * y, axis=0, keepdims=True)
    st_ref[...] = jnp.concatenate([ps[None], pq[None]], axis=1)


def _conv_mm_call(h, acc, dga, dgb, W0, W1, b):
    N, Din = h.shape
    Dout = W0.shape[1]
    R = 500
    G = N // R
    full = lambda shp: pl.BlockSpec(shp, lambda i: (0,) * len(shp))
    blk3 = lambda r, d: pl.BlockSpec((1, r, d), lambda i: (i, 0, 0))
    y, st = pl.pallas_call(
        _conv_mm_body, grid=(G,),
        in_specs=[blk3(R, Din), blk3(R, Din), blk3(R, 1), blk3(R, 1),
                  full(W0.shape), full(W1.shape), full((1, Dout))],
        out_specs=[blk3(R, Dout),
                   pl.BlockSpec((1, 2, Dout), lambda i: (i, 0, 0))],
        out_shape=[
            jax.ShapeDtypeStruct((G, R, Dout), jnp.float32),
            jax.ShapeDtypeStruct((G, 2, Dout), jnp.float32),
        ])(h.reshape(G, R, Din), acc.reshape(G, R, Din),
           dga.reshape(G, R, 1), dgb.reshape(G, R, 1), W0, W1, b.reshape(1, -1))
    return y.reshape(N, Dout), st


def _norm_body_with_next(y_ref, st_ref, dga_ref, dgb_ref, g_ref, bb_ref,
                         wn_ref, h1_ref, un_ref, *, n_rows):
    st = jnp.sum(st_ref[...], axis=0)
    mu = st[0:1] / n_rows
    msq = st[1:2] / n_rows
    var = msq - mu * mu
    z = (y_ref[0] - mu) / jnp.sqrt(var + 1e-5) * g_ref[0] + bb_ref[0]
    h1 = _leaky(z)
    h1_ref[0] = h1
    dis = _dis_of(dga_ref[0] + dgb_ref[0])
    un_ref[0] = dis * jnp.dot(h1, wn_ref[...], preferred_element_type=jnp.float32)


def _norm_next_call(y, st, dga, dgb, g, bb, Wn):
    N, Dout = y.shape
    R = 500
    G = N // R
    Dn = Wn.shape[1]
    full = lambda shp: pl.BlockSpec(shp, lambda i: (0,) * len(shp))
    blk3 = lambda r, d: pl.BlockSpec((1, r, d), lambda i: (i, 0, 0))
    body = functools.partial(_norm_body_with_next, n_rows=float(N))
    h1, un = pl.pallas_call(
        body, grid=(G,),
        in_specs=[blk3(R, Dout), full(st.shape), blk3(R, 1), blk3(R, 1),
                  full((1, Dout)), full((1, Dout)), full(Wn.shape)],
        out_specs=[blk3(R, Dout), blk3(R, Dn)],
        out_shape=[
            jax.ShapeDtypeStruct((G, R, Dout), jnp.float32),
            jax.ShapeDtypeStruct((G, R, Dn), jnp.float32),
        ])(y.reshape(G, R, Dout), st, dga.reshape(G, R, 1),
           dgb.reshape(G, R, 1), g.reshape(1, -1), bb.reshape(1, -1), Wn)
    return h1.reshape(N, Dout), un.reshape(N, Dn)


def _norm_body_final(y_ref, st_ref, g_ref, bb_ref, o_ref, *, n_rows):
    st = jnp.sum(st_ref[...], axis=0)
    mu = st[0:1] / n_rows
    msq = st[1:2] / n_rows
    var = msq - mu * mu
    z = (y_ref[0] - mu) / jnp.sqrt(var + 1e-5) * g_ref[0] + bb_ref[0]
    o_ref[0] = _leaky(z)


def _norm_final_call(y, st, g, bb):
    N, Dout = y.shape
    R = 500
    G = N // R
    full = lambda shp: pl.BlockSpec(shp, lambda i: (0,) * len(shp))
    blk3 = lambda r, d: pl.BlockSpec((1, r, d), lambda i: (i, 0, 0))
    body = functools.partial(_norm_body_final, n_rows=float(N))
    out = pl.pallas_call(
        body, grid=(G,),
        in_specs=[blk3(R, Dout), full(st.shape), full((1, Dout)), full((1, Dout))],
        out_specs=blk3(R, Dout),
        out_shape=jax.ShapeDtypeStruct((G, R, Dout), jnp.float32),
    )(y.reshape(G, R, Dout), st, g.reshape(1, -1), bb.reshape(1, -1))
    return out.reshape(N, Dout)


def _conv2_mm_body(h_ref, aa_ref, ab_ref, dga_ref, dgb_ref, w0_ref, b_ref,
                   y_ref, st_ref):
    dis = _dis_of(dga_ref[0] + dgb_ref[0])
    d = w0_ref.shape[1]
    ax = dis * (aa_ref[0][:, :d] + ab_ref[0][:, :d])
    y = (jnp.dot(h_ref[0], w0_ref[...], preferred_element_type=jnp.float32)
         - ax + b_ref[0])
    y_ref[0] = y
    ps = jnp.sum(y, axis=0, keepdims=True)
    pq = jnp.sum(y * y, axis=0, keepdims=True)
    st_ref[...] = jnp.concatenate([ps[None], pq[None]], axis=1)


def _conv2_mm_call(h1, acc2a, acc2b, dga, dgb, W0, b):
    N, Din = h1.shape
    Dout = W0.shape[1]
    R = 500
    G = N // R
    full = lambda shp: pl.BlockSpec(shp, lambda i: (0,) * len(shp))
    blk3 = lambda r, d: pl.BlockSpec((1, r, d), lambda i: (i, 0, 0))
    y, st = pl.pallas_call(
        _conv2_mm_body, grid=(G,),
        in_specs=[blk3(R, Din), blk3(R, 128), blk3(R, 128),
                  blk3(R, 1), blk3(R, 1), full(W0.shape), full((1, Dout))],
        out_specs=[blk3(R, Dout),
                   pl.BlockSpec((1, 2, Dout), lambda i: (i, 0, 0))],
        out_shape=[
            jax.ShapeDtypeStruct((G, R, Dout), jnp.float32),
            jax.ShapeDtypeStruct((G, 2, Dout), jnp.float32),
        ])(h1.reshape(G, R, Din), acc2a.reshape(G, R, 128),
           acc2b.reshape(G, R, 128), dga.reshape(G, R, 1),
           dgb.reshape(G, R, 1), W0, b.reshape(1, -1))
    return y.reshape(N, Dout), st


def kernel(x, style_vector, edge_index, edge_attr, batch_size, nroi, params):
    N, D = x.shape
    Bn = style_vector.shape[0]
    R = N // Bn

    E = edge_index.shape[1]
    src = edge_index[0].astype(jnp.int32)
    dst = edge_index[1].astype(jnp.int32)
    ew = edge_attr.astype(jnp.float32)
    nw = _NC * _NS
    gsz = _GR * _CH
    src32 = src.reshape(nw, E // (nw * gsz), _GR, _CH)
    dst32 = dst.reshape(nw, E // (nw * gsz), _GR, _CH)
    ew32 = ew.reshape(nw, E // (nw * gsz), _GR, _CH)
    src16 = src.reshape(_NS, E // (_NS * gsz), _GR, _CH)
    dst16 = dst.reshape(_NS, E // (_NS * gsz), _GR, _CH)
    ew16 = ew.reshape(_NS, E // (_NS * gsz), _GR, _CH)
    srcd = src.reshape(nw, E // (nw * _CH), _CH)
    ewd = ew.reshape(nw, E // (nw * _CH), _CH)

    dg0, dg1 = _deg_call(srcd, ewd)
    dga = dg0[:N].reshape(N, 1)
    dgb = dg1[:N].reshape(N, 1)

    h, u0, u1, u2, u3 = _mlp_call(x, style_vector, dga, dgb, params)

    a0, a1, a2, a3 = _acc1_call((u0, u1, u2, u3), src16, dst16, ew16)
    acc1 = jnp.concatenate(
        [a0[:N], a1[:N], a2[:N], a3[:N]], axis=1)

    y1, st1 = _conv_mm_call(h, acc1, dga, dgb,
                            params['conv1_W0'], params['conv1_W1'],
                            params['conv1_b'])
    w1n = jnp.pad(params['conv2_W1'], ((0, 0), (0, 64)))
    h1, u2w = _norm_next_call(y1, st1, dga, dgb,
                              params['conv1_g'], params['conv1_bb'], w1n)

    b0, b1 = _acc2_call(u2w, src32, dst32, ew32)

    y2, st2 = _conv2_mm_call(h1, b0[:N], b1[:N], dga, dgb,
                             params['conv2_W0'], params['conv2_b'])
    out = _norm_final_call(y2, st2, params['conv2_g'], params['conv2_bb'])
    return out.reshape(Bn, R, out.shape[1])


# speedup vs baseline: 13.6767x; 1.0101x over previous
import functools

import jax
import jax.numpy as jnp
from jax import lax
from jax.experimental import pallas as pl
from jax.experimental.pallas import tpu as pltpu
from jax.experimental.pallas import tpu_sc as plsc

_NC = 2
_NS = 16
_NPAD = 10240
_CH = 80


def _leaky(v):
    return jnp.where(v > 0, v, 0.2 * v)


def _dis_of(deg):
    return jnp.where(deg > 0, 1.0 / jnp.sqrt(deg + 1e-12), 0.0)


def _deg_body(src_hbm, ew_hbm, deg0_hbm, deg1_hbm, sidx, ewv, z_v, acc_sh, sem):
    c = lax.axis_index("c")
    s = lax.axis_index("s")
    nrow = sidx.shape[0]
    sl = _NPAD // _NS

    def zb(i, _):
        z_v[pl.ds(i * 16, 16)] = jnp.zeros((16,), jnp.float32)
        return 0
    lax.fori_loop(0, sl // 16, zb, 0)
    pltpu.sync_copy(z_v, acc_sh.at[pl.ds(s * sl, sl)])

    wid = c * _NS + s
    pltpu.sync_copy(src_hbm.at[wid], sidx)
    pltpu.sync_copy(ew_hbm.at[wid], ewv)
    plsc.subcore_barrier()

    def body(i, _):
        pltpu.sync_copy(ewv.at[i], acc_sh.at[sidx.at[i]], add=True)
        return 0
    lax.fori_loop(0, nrow, body, 0)
    plsc.subcore_barrier()

    @pl.when(c == 0)
    def _():
        pltpu.sync_copy(acc_sh.at[pl.ds(s * sl, sl)], deg0_hbm.at[pl.ds(s * sl, sl)])

    @pl.when(c == 1)
    def _():
        pltpu.sync_copy(acc_sh.at[pl.ds(s * sl, sl)], deg1_hbm.at[pl.ds(s * sl, sl)])


def _deg_call(src2d, ew2d):
    nrow = src2d.shape[1]
    mesh = plsc.VectorSubcoreMesh(core_axis_name="c", subcore_axis_name="s")
    f = pl.kernel(
        _deg_body,
        out_type=[jax.ShapeDtypeStruct((_NPAD,), jnp.float32),
                  jax.ShapeDtypeStruct((_NPAD,), jnp.float32)],
        mesh=mesh,
        scratch_types=[
            pltpu.VMEM((nrow, _CH), jnp.int32),
            pltpu.VMEM((nrow, _CH), jnp.float32),
            pltpu.VMEM((_NPAD // _NS,), jnp.float32),
            pltpu.VMEM_SHARED((_NPAD,), jnp.float32),
            pltpu.SemaphoreType.DMA,
        ],
    )
    return f(src2d, ew2d)


_GR = 25


def _zero_rows(buf, nj):
    def zb(k, _):
        for j in range(nj):
            buf[k, pl.ds(j * 16, 16)] = jnp.zeros((16,), jnp.float32)
        return 0
    lax.fori_loop(0, _CH, zb, 0)


def _zero_acc(acc_sh, zrows, s):
    sl = _NPAD // _NS

    def z(i, _):
        pltpu.sync_copy(zrows, acc_sh.at[pl.ds(s * sl + i * _CH, _CH)])
        return 0
    lax.fori_loop(0, sl // _CH, z, 0)


def _writeout(acc_sh, out_hbm, buf, s):
    sl = _NPAD // _NS

    def w(i, _):
        pltpu.sync_copy(acc_sh.at[pl.ds(s * sl + i * _CH, _CH)], buf)
        pltpu.sync_copy(buf, out_hbm.at[pl.ds(s * sl + i * _CH, _CH)])
        return 0
    lax.fori_loop(0, sl // _CH, w, 0)


def _edge_sweeps(u_hbm, acc_sh, src4, dst4, ew4, tid,
                 sidx, didx, ewv, rbufs, gsems, ssems, nj):
    nswp = src4.shape[1]
    ntri = _GR // 3

    def scale(rows, wi):
        def grp(g, _):
            w16 = ewv[wi, pl.ds(g * 16, 16)]
            for l in range(16):
                w = w16[l]
                k = g * 16 + l
                for j in range(nj):
                    rows[k, pl.ds(j * 16, 16)] = rows[k, pl.ds(j * 16, 16)] * w
            return 0
        lax.fori_loop(0, _CH // 16, grp, 0)

    r0, r1, r2 = rbufs
    g0, g1, g2 = gsems
    s0, s1, s2 = ssems

    def sweep(g, _):
        pltpu.sync_copy(src4.at[tid, g], sidx)
        pltpu.sync_copy(dst4.at[tid, g], didx)
        pltpu.sync_copy(ew4.at[tid, g], ewv)
        for b in range(3):
            pltpu.async_copy(u_hbm.at[sidx.at[b]], rbufs[b], gsems[b])

        def body(i, _):
            for b in range(3):
                k = 3 * i + b
                pltpu.make_async_copy(u_hbm.at[sidx.at[0]], rbufs[b], gsems[b]).wait()
                scale(rbufs[b], k)
                pltpu.async_copy(rbufs[b], acc_sh.at[didx.at[k]], ssems[b], add=True)

                @pl.when(k + 3 <= _GR - 1)
                def _(b=b, k=k):
                    pltpu.make_async_copy(rbufs[b], acc_sh.at[didx.at[k]], ssems[b]).wait()
                    pltpu.async_copy(u_hbm.at[sidx.at[k + 3]], rbufs[b], gsems[b])
            return 0
        lax.fori_loop(0, ntri, body, 0)
        pltpu.make_async_copy(u_hbm.at[sidx.at[0]], r0, g0).wait()
        scale(r0, _GR - 1)
        pltpu.async_copy(r0, acc_sh.at[didx.at[_GR - 1]], s0, add=True)
        pltpu.make_async_copy(r0, acc_sh.at[didx.at[_GR - 1]], s0).wait()
        pltpu.make_async_copy(r1, acc_sh.at[didx.at[_GR - 3]], s1).wait()
        pltpu.make_async_copy(r2, acc_sh.at[didx.at[_GR - 2]], s2).wait()
        return 0
    lax.fori_loop(0, nswp, sweep, 0)


def _acc1_body(u0, u1, u2, u3, src4, dst4, ew4,
               a0, a1, a2, a3,
               sidx, didx, ewv, r0, r1, r2, acc_sh,
               g0, g1, g2, s0, s1, s2):
    c = lax.axis_index("c")
    s = lax.axis_index("s")
    u_all = (u0, u1, u2, u3)
    out_all = (a0, a1, a2, a3)
    for p in range(4):
        @pl.when(c == p // 2)
        def _(p=p):
            _zero_rows(r0, 8)
            _zero_acc(acc_sh, r0, s)
            plsc.subcore_barrier()
            _edge_sweeps(u_all[p], acc_sh, src4, dst4, ew4, s,
                         sidx, didx, ewv, (r0, r1, r2),
                         (g0, g1, g2), (s0, s1, s2), 8)
            plsc.subcore_barrier()
            _writeout(acc_sh, out_all[p], r0, s)
            plsc.subcore_barrier()


def _acc1_call(u_panels, src4, dst4, ew4):
    mesh = plsc.VectorSubcoreMesh(core_axis_name="c", subcore_axis_name="s")
    f = pl.kernel(
        _acc1_body,
        out_type=[jax.ShapeDtypeStruct((_NPAD, 128), jnp.float32)] * 4,
        mesh=mesh,
        scratch_types=[
            pltpu.VMEM((_GR, _CH), jnp.int32),
            pltpu.VMEM((_GR, _CH), jnp.int32),
            pltpu.VMEM((_GR, _CH), jnp.float32),
            pltpu.VMEM((_CH, 128), jnp.float32),
            pltpu.VMEM((_CH, 128), jnp.float32),
            pltpu.VMEM((_CH, 128), jnp.float32),
            pltpu.VMEM_SHARED((_NPAD, 128), jnp.float32),
            pltpu.SemaphoreType.DMA,
            pltpu.SemaphoreType.DMA,
            pltpu.SemaphoreType.DMA,
            pltpu.SemaphoreType.DMA,
            pltpu.SemaphoreType.DMA,
            pltpu.SemaphoreType.DMA,
        ],
    )
    return f(*u_panels, src4, dst4, ew4)


def _acc2_body(u_hbm, src4, dst4, ew4, b0, b1,
               sidx, didx, ewv, r0, r1, r2, acc_sh,
               g0, g1, g2, s0, s1, s2):
    c = lax.axis_index("c")
    s = lax.axis_index("s")
    wid = c * _NS + s
    _zero_rows(r0, 8)
    _zero_acc(acc_sh, r0, s)
    plsc.subcore_barrier()
    _edge_sweeps(u_hbm, acc_sh, src4, dst4, ew4, wid,
                 sidx, didx, ewv, (r0, r1, r2),
                 (g0, g1, g2), (s0, s1, s2), 8)
    plsc.subcore_barrier()

    @pl.when(c == 0)
    def _():
        _writeout(acc_sh, b0, r0, s)

    @pl.when(c == 1)
    def _():
        _writeout(acc_sh, b1, r0, s)


def _acc2_call(u2, src4, dst4, ew4):
    mesh = plsc.VectorSubcoreMesh(core_axis_name="c", subcore_axis_name="s")
    f = pl.kernel(
        _acc2_body,
        out_type=[jax.ShapeDtypeStruct((_NPAD, 128), jnp.float32)] * 2,
        mesh=mesh,
        scratch_types=[
            pltpu.VMEM((_GR, _CH), jnp.int32),
            pltpu.VMEM((_GR, _CH), jnp.int32),
            pltpu.VMEM((_GR, _CH), jnp.float32),
            pltpu.VMEM((_CH, 128), jnp.float32),
            pltpu.VMEM((_CH, 128), jnp.float32),
            pltpu.VMEM((_CH, 128), jnp.float32),
            pltpu.VMEM_SHARED((_NPAD, 128), jnp.float32),
            pltpu.SemaphoreType.DMA,
            pltpu.SemaphoreType.DMA,
            pltpu.SemaphoreType.DMA,
            pltpu.SemaphoreType.DMA,
            pltpu.SemaphoreType.DMA,
            pltpu.SemaphoreType.DMA,
        ],
    )
    return f(u2, src4, dst4, ew4)


def _mlp_body(x_ref, sty_ref, dga_ref, dgb_ref,
              w1, b1, ws1, bs1, w2, b2, ws2, bs2,
              w3, b3, ws3, bs3, w4, b4, ws4, bs4,
              h_ref, u0_ref, u1_ref, u2_ref, u3_ref):
    h = x_ref[0]
    sty = sty_ref[0]

    def unit(h, W, b, Ws, bs):
        d = W.shape[1]
        y = jnp.dot(h, W, preferred_element_type=jnp.float32) + b[0]
        mu = jnp.mean(y, axis=0, keepdims=True)
        var = jnp.mean((y - mu) * (y - mu), axis=0, keepdims=True)
        yn = (y - mu) / jnp.sqrt(var + 1e-5)
        gb = jnp.dot(sty, Ws, preferred_element_type=jnp.float32) + bs[0]
        gamma = gb[:, :d]
        beta = gb[:, d:]
        return _leaky((1.0 + gamma) * yn + beta)

    h = unit(h, w1[...], b1[...], ws1[...], bs1[...])
    h = unit(h, w2[...], b2[...], ws2[...], bs2[...])
    h = unit(h, w3[...], b3[...], ws3[...], bs3[...])
    h = unit(h, w4[...], b4[...], ws4[...], bs4[...])
    h_ref[0] = h
    dis = _dis_of(dga_ref[0] + dgb_ref[0])
    u = dis * h
    u0_ref[0] = u[:, 0:128]
    u1_ref[0] = u[:, 128:256]
    u2_ref[0] = u[:, 256:384]
    u3_ref[0] = u[:, 384:512]


def _mlp_call(x, sty, dga, dgb, params):
    N, D = x.shape
    Bn = sty.shape[0]
    R = N // Bn
    full = lambda shp: pl.BlockSpec(shp, lambda i: (0,) * len(shp))
    blk3 = lambda r, d: pl.BlockSpec((1, r, d), lambda i: (i, 0, 0))
    w_args = []
    w_specs = []
    for li in range(1, 5):
        W = params['fc%d_W' % li]
        b = params['fc%d_b' % li].reshape(1, -1)
        Ws = params['fc%d_Ws' % li]
        bs = params['fc%d_bs' % li].reshape(1, -1)
        w_args += [W, b, Ws, bs]
        w_specs += [full(W.shape), full(b.shape), full(Ws.shape), full(bs.shape)]
    out_shape = [jax.ShapeDtypeStruct((Bn, R, 512), jnp.float32)] + [
        jax.ShapeDtypeStruct((Bn, R, 128), jnp.float32)] * 4
    in_specs = [blk3(R, D), pl.BlockSpec((1, 1, sty.shape[1]), lambda i: (i, 0, 0)),
                blk3(R, 1), blk3(R, 1)] + w_specs
    out_specs = [blk3(R, 512)] + [blk3(R, 128)] * 4
    outs = pl.pallas_call(
        _mlp_body, grid=(Bn,), in_specs=in_specs, out_specs=out_specs,
        out_shape=out_shape)(
            x.reshape(Bn, R, D), sty.reshape(Bn, 1, -1),
            dga.reshape(Bn, R, 1), dgb.reshape(Bn, R, 1), *w_args)
    return tuple(o.reshape(N, -1) for o in outs)


def _conv_mm_body(h_ref, a0_ref, a1_ref, a2_ref, a3_ref,
                  dga_ref, dgb_ref, w0_ref, w1_ref, b_ref,
                  y_ref, st_ref):
    dis = _dis_of(dga_ref[0] + dgb_ref[0])
    ax = dis * jnp.concatenate(
        [a0_ref[0], a1_ref[0], a2_ref[0], a3_ref[0]], axis=1)
    y = (jnp.dot(h_ref[0], w0_ref[...], preferred_element_type=jnp.float32)
         - jnp.dot(ax, w1_ref[...], preferred_element_type=jnp.float32)
         + b_ref[0])
    y_ref[0] = y
    ps = jnp.sum(y, axis=0, keepdims=True)
    pq = jnp.sum(y * y, axis=0, keepdims=True)
    st_ref[...] = jnp.concatenate([ps[None], pq[None]], axis=1)


def _conv_mm_call(h, panels, dga, dgb, W0, W1, b):
    N, Din = h.shape
    Dout = W0.shape[1]
    R = 500
    G = N // R
    full = lambda shp: pl.BlockSpec(shp, lambda i: (0,) * len(shp))
    blk3 = lambda r, d: pl.BlockSpec((1, r, d), lambda i: (i, 0, 0))
    y, st = pl.pallas_call(
        _conv_mm_body, grid=(G,),
        in_specs=[blk3(R, Din)] + [blk3(R, 128)] * 4 +
                 [blk3(R, 1), blk3(R, 1),
                  full(W0.shape), full(W1.shape), full((1, Dout))],
        out_specs=[blk3(R, Dout),
                   pl.BlockSpec((1, 2, Dout), lambda i: (i, 0, 0))],
        out_shape=[
            jax.ShapeDtypeStruct((G, R, Dout), jnp.float32),
            jax.ShapeDtypeStruct((G, 2, Dout), jnp.float32),
        ])(h.reshape(G, R, Din),
           *[p[:N].reshape(G, R, 128) for p in panels],
           dga.reshape(G, R, 1), dgb.reshape(G, R, 1), W0, W1, b.reshape(1, -1))
    return y.reshape(N, Dout), st


def _norm_body_with_next(y_ref, st_ref, dga_ref, dgb_ref, g_ref, bb_ref,
                         wn_ref, h1_ref, un_ref, *, n_rows):
    st = jnp.sum(st_ref[...], axis=0)
    mu = st[0:1] / n_rows
    msq = st[1:2] / n_rows
    var = msq - mu * mu
    z = (y_ref[0] - mu) / jnp.sqrt(var + 1e-5) * g_ref[0] + bb_ref[0]
    h1 = _leaky(z)
    h1_ref[0] = h1
    dis = _dis_of(dga_ref[0] + dgb_ref[0])
    un_ref[0] = dis * jnp.dot(h1, wn_ref[...], preferred_element_type=jnp.float32)


def _norm_next_call(y, st, dga, dgb, g, bb, Wn):
    N, Dout = y.shape
    R = 500
    G = N // R
    Dn = Wn.shape[1]
    full = lambda shp: pl.BlockSpec(shp, lambda i: (0,) * len(shp))
    blk3 = lambda r, d: pl.BlockSpec((1, r, d), lambda i: (i, 0, 0))
    body = functools.partial(_norm_body_with_next, n_rows=float(N))
    h1, un = pl.pallas_call(
        body, grid=(G,),
        in_specs=[blk3(R, Dout), full(st.shape), blk3(R, 1), blk3(R, 1),
                  full((1, Dout)), full((1, Dout)), full(Wn.shape)],
        out_specs=[blk3(R, Dout), blk3(R, Dn)],
        out_shape=[
            jax.ShapeDtypeStruct((G, R, Dout), jnp.float32),
            jax.ShapeDtypeStruct((G, R, Dn), jnp.float32),
        ])(y.reshape(G, R, Dout), st, dga.reshape(G, R, 1),
           dgb.reshape(G, R, 1), g.reshape(1, -1), bb.reshape(1, -1), Wn)
    return h1.reshape(N, Dout), un.reshape(N, Dn)


def _norm_body_final(y_ref, st_ref, g_ref, bb_ref, o_ref, *, n_rows):
    st = jnp.sum(st_ref[...], axis=0)
    mu = st[0:1] / n_rows
    msq = st[1:2] / n_rows
    var = msq - mu * mu
    z = (y_ref[0] - mu) / jnp.sqrt(var + 1e-5) * g_ref[0] + bb_ref[0]
    o_ref[0] = _leaky(z)


def _norm_final_call(y, st, g, bb):
    N, Dout = y.shape
    R = 500
    G = N // R
    full = lambda shp: pl.BlockSpec(shp, lambda i: (0,) * len(shp))
    blk3 = lambda r, d: pl.BlockSpec((1, r, d), lambda i: (i, 0, 0))
    body = functools.partial(_norm_body_final, n_rows=float(N))
    out = pl.pallas_call(
        body, grid=(G,),
        in_specs=[blk3(R, Dout), full(st.shape), full((1, Dout)), full((1, Dout))],
        out_specs=blk3(R, Dout),
        out_shape=jax.ShapeDtypeStruct((G, R, Dout), jnp.float32),
    )(y.reshape(G, R, Dout), st, g.reshape(1, -1), bb.reshape(1, -1))
    return out.reshape(N, Dout)


def _conv2_mm_body(h_ref, aa_ref, ab_ref, dga_ref, dgb_ref, w0_ref, b_ref,
                   y_ref, st_ref):
    dis = _dis_of(dga_ref[0] + dgb_ref[0])
    d = w0_ref.shape[1]
    ax = dis * (aa_ref[0][:, :d] + ab_ref[0][:, :d])
    y = (jnp.dot(h_ref[0], w0_ref[...], preferred_element_type=jnp.float32)
         - ax + b_ref[0])
    y_ref[0] = y
    ps = jnp.sum(y, axis=0, keepdims=True)
    pq = jnp.sum(y * y, axis=0, keepdims=True)
    st_ref[...] = jnp.concatenate([ps[None], pq[None]], axis=1)


def _conv2_mm_call(h1, acc2a, acc2b, dga, dgb, W0, b):
    N, Din = h1.shape
    Dout = W0.shape[1]
    R = 500
    G = N // R
    full = lambda shp: pl.BlockSpec(shp, lambda i: (0,) * len(shp))
    blk3 = lambda r, d: pl.BlockSpec((1, r, d), lambda i: (i, 0, 0))
    y, st = pl.pallas_call(
        _conv2_mm_body, grid=(G,),
        in_specs=[blk3(R, Din), blk3(R, 128), blk3(R, 128),
                  blk3(R, 1), blk3(R, 1), full(W0.shape), full((1, Dout))],
        out_specs=[blk3(R, Dout),
                   pl.BlockSpec((1, 2, Dout), lambda i: (i, 0, 0))],
        out_shape=[
            jax.ShapeDtypeStruct((G, R, Dout), jnp.float32),
            jax.ShapeDtypeStruct((G, 2, Dout), jnp.float32),
        ])(h1.reshape(G, R, Din), acc2a.reshape(G, R, 128),
           acc2b.reshape(G, R, 128), dga.reshape(G, R, 1),
           dgb.reshape(G, R, 1), W0, b.reshape(1, -1))
    return y.reshape(N, Dout), st


def kernel(x, style_vector, edge_index, edge_attr, batch_size, nroi, params):
    N, D = x.shape
    Bn = style_vector.shape[0]
    R = N // Bn

    E = edge_index.shape[1]
    src = edge_index[0].astype(jnp.int32)
    dst = edge_index[1].astype(jnp.int32)
    ew = edge_attr.astype(jnp.float32)
    nw = _NC * _NS
    gsz = _GR * _CH
    src32 = src.reshape(nw, E // (nw * gsz), _GR, _CH)
    dst32 = dst.reshape(nw, E // (nw * gsz), _GR, _CH)
    ew32 = ew.reshape(nw, E // (nw * gsz), _GR, _CH)
    src16 = src.reshape(_NS, E // (_NS * gsz), _GR, _CH)
    dst16 = dst.reshape(_NS, E // (_NS * gsz), _GR, _CH)
    ew16 = ew.reshape(_NS, E // (_NS * gsz), _GR, _CH)
    srcd = src.reshape(nw, E // (nw * _CH), _CH)
    ewd = ew.reshape(nw, E // (nw * _CH), _CH)

    dg0, dg1 = _deg_call(srcd, ewd)
    dga = dg0[:N].reshape(N, 1)
    dgb = dg1[:N].reshape(N, 1)

    h, u0, u1, u2, u3 = _mlp_call(x, style_vector, dga, dgb, params)

    a0, a1, a2, a3 = _acc1_call((u0, u1, u2, u3), src16, dst16, ew16)

    y1, st1 = _conv_mm_call(h, (a0, a1, a2, a3), dga, dgb,
                            params['conv1_W0'], params['conv1_W1'],
                            params['conv1_b'])
    w1n = jnp.pad(params['conv2_W1'], ((0, 0), (0, 64)))
    h1, u2w = _norm_next_call(y1, st1, dga, dgb,
                              params['conv1_g'], params['conv1_bb'], w1n)

    b0, b1 = _acc2_call(u2w, src32, dst32, ew32)

    y2, st2 = _conv2_mm_call(h1, b0[:N], b1[:N], dga, dgb,
                             params['conv2_W0'], params['conv2_b'])
    out = _norm_final_call(y2, st2, params['conv2_g'], params['conv2_bb'])
    return out.reshape(Bn, R, out.shape[1])


# speedup vs baseline: 14.0994x; 1.0309x over previous
import functools

import jax
import jax.numpy as jnp
from jax import lax
from jax.experimental import pallas as pl
from jax.experimental.pallas import tpu as pltpu
from jax.experimental.pallas import tpu_sc as plsc

_NC = 2
_NS = 16
_NPAD = 10240
_CH = 80


def _leaky(v):
    return jnp.where(v > 0, v, 0.2 * v)


def _dis_of(deg):
    return jnp.where(deg > 0, 1.0 / jnp.sqrt(deg + 1e-12), 0.0)


def _deg_body(src_hbm, ew_hbm, deg0_hbm, deg1_hbm, sidx, ewv, z_v, acc_sh, sem):
    c = lax.axis_index("c")
    s = lax.axis_index("s")
    nrow = sidx.shape[0]
    sl = _NPAD // _NS

    def zb(i, _):
        z_v[pl.ds(i * 16, 16)] = jnp.zeros((16,), jnp.float32)
        return 0
    lax.fori_loop(0, sl // 16, zb, 0)
    pltpu.sync_copy(z_v, acc_sh.at[pl.ds(s * sl, sl)])

    wid = c * _NS + s
    pltpu.sync_copy(src_hbm.at[wid], sidx)
    pltpu.sync_copy(ew_hbm.at[wid], ewv)
    plsc.subcore_barrier()

    def body(i, _):
        pltpu.sync_copy(ewv.at[i], acc_sh.at[sidx.at[i]], add=True)
        return 0
    lax.fori_loop(0, nrow, body, 0)
    plsc.subcore_barrier()

    @pl.when(c == 0)
    def _():
        pltpu.sync_copy(acc_sh.at[pl.ds(s * sl, sl)], deg0_hbm.at[pl.ds(s * sl, sl)])

    @pl.when(c == 1)
    def _():
        pltpu.sync_copy(acc_sh.at[pl.ds(s * sl, sl)], deg1_hbm.at[pl.ds(s * sl, sl)])


def _deg_call(src2d, ew2d):
    nrow = src2d.shape[1]
    mesh = plsc.VectorSubcoreMesh(core_axis_name="c", subcore_axis_name="s")
    f = pl.kernel(
        _deg_body,
        out_type=[jax.ShapeDtypeStruct((_NPAD,), jnp.float32),
                  jax.ShapeDtypeStruct((_NPAD,), jnp.float32)],
        mesh=mesh,
        scratch_types=[
            pltpu.VMEM((nrow, _CH), jnp.int32),
            pltpu.VMEM((nrow, _CH), jnp.float32),
            pltpu.VMEM((_NPAD // _NS,), jnp.float32),
            pltpu.VMEM_SHARED((_NPAD,), jnp.float32),
            pltpu.SemaphoreType.DMA,
        ],
    )
    return f(src2d, ew2d)


_GR = 25


def _zero_rows(buf, nj):
    def zb(k, _):
        for j in range(nj):
            buf[k, pl.ds(j * 16, 16)] = jnp.zeros((16,), jnp.float32)
        return 0
    lax.fori_loop(0, _CH, zb, 0)


def _zero_acc(acc_sh, zrows, s):
    sl = _NPAD // _NS

    def z(i, _):
        pltpu.sync_copy(zrows, acc_sh.at[pl.ds(s * sl + i * _CH, _CH)])
        return 0
    lax.fori_loop(0, sl // _CH, z, 0)


def _writeout(acc_sh, out_hbm, buf, s):
    sl = _NPAD // _NS

    def w(i, _):
        pltpu.sync_copy(acc_sh.at[pl.ds(s * sl + i * _CH, _CH)], buf)
        pltpu.sync_copy(buf, out_hbm.at[pl.ds(s * sl + i * _CH, _CH)])
        return 0
    lax.fori_loop(0, sl // _CH, w, 0)


def _edge_sweeps(u_hbm, acc_sh, epk, tid,
                 ebuf, rbufs, gsems, ssems, nj):
    nswp = epk.shape[1]
    ntri = _GR // 3

    def scale(rows, wi):
        def grp(g, _):
            w16 = jax.lax.bitcast_convert_type(ebuf[2, wi, pl.ds(g * 16, 16)], jnp.float32)
            for l in range(16):
                w = w16[l]
                k = g * 16 + l
                for j in range(nj):
                    rows[k, pl.ds(j * 16, 16)] = rows[k, pl.ds(j * 16, 16)] * w
            return 0
        lax.fori_loop(0, _CH // 16, grp, 0)

    r0, r1, r2 = rbufs
    g0, g1, g2 = gsems
    s0, s1, s2 = ssems

    def sweep(g, _):
        pltpu.sync_copy(epk.at[tid, g], ebuf)
        for b in range(3):
            pltpu.async_copy(u_hbm.at[ebuf.at[0, b]], rbufs[b], gsems[b])

        def body(i, _):
            for b in range(3):
                k = 3 * i + b
                pltpu.make_async_copy(u_hbm.at[ebuf.at[0, 0]], rbufs[b], gsems[b]).wait()
                scale(rbufs[b], k)
                pltpu.async_copy(rbufs[b], acc_sh.at[ebuf.at[1, k]], ssems[b], add=True)

                @pl.when(k + 3 <= _GR - 1)
                def _(b=b, k=k):
                    pltpu.make_async_copy(rbufs[b], acc_sh.at[ebuf.at[1, 0]], ssems[b]).wait()
                    pltpu.async_copy(u_hbm.at[ebuf.at[0, k + 3]], rbufs[b], gsems[b])
            return 0
        lax.fori_loop(0, ntri, body, 0)
        pltpu.make_async_copy(u_hbm.at[ebuf.at[0, 0]], r0, g0).wait()
        scale(r0, _GR - 1)
        pltpu.async_copy(r0, acc_sh.at[ebuf.at[1, _GR - 1]], s0, add=True)
        pltpu.make_async_copy(r0, acc_sh.at[ebuf.at[1, 0]], s0).wait()
        pltpu.make_async_copy(r1, acc_sh.at[ebuf.at[1, 0]], s1).wait()
        pltpu.make_async_copy(r2, acc_sh.at[ebuf.at[1, 0]], s2).wait()
        return 0
    lax.fori_loop(0, nswp, sweep, 0)


def _acc1_body(u0, u1, u2, u3, epk,
               a0, a1, a2, a3,
               ebuf, r0, r1, r2, acc_sh,
               g0, g1, g2, s0, s1, s2):
    c = lax.axis_index("c")
    s = lax.axis_index("s")
    u_all = (u0, u1, u2, u3)
    out_all = (a0, a1, a2, a3)
    for p in range(4):
        @pl.when(c == p // 2)
        def _(p=p):
            _zero_rows(r0, 8)
            _zero_acc(acc_sh, r0, s)
            plsc.subcore_barrier()
            _edge_sweeps(u_all[p], acc_sh, epk, s,
                         ebuf, (r0, r1, r2),
                         (g0, g1, g2), (s0, s1, s2), 8)
            plsc.subcore_barrier()
            _writeout(acc_sh, out_all[p], r0, s)
            plsc.subcore_barrier()


def _acc1_call(u_panels, epk):
    mesh = plsc.VectorSubcoreMesh(core_axis_name="c", subcore_axis_name="s")
    f = pl.kernel(
        _acc1_body,
        out_type=[jax.ShapeDtypeStruct((_NPAD, 128), jnp.float32)] * 4,
        mesh=mesh,
        scratch_types=[
            pltpu.VMEM((3, _GR, _CH), jnp.int32),
            pltpu.VMEM((_CH, 128), jnp.float32),
            pltpu.VMEM((_CH, 128), jnp.float32),
            pltpu.VMEM((_CH, 128), jnp.float32),
            pltpu.VMEM_SHARED((_NPAD, 128), jnp.float32),
            pltpu.SemaphoreType.DMA,
            pltpu.SemaphoreType.DMA,
            pltpu.SemaphoreType.DMA,
            pltpu.SemaphoreType.DMA,
            pltpu.SemaphoreType.DMA,
            pltpu.SemaphoreType.DMA,
        ],
    )
    return f(*u_panels, epk)


def _acc2_body(u_hbm, epk, b0, b1,
               ebuf, r0, r1, r2, acc_sh,
               g0, g1, g2, s0, s1, s2):
    c = lax.axis_index("c")
    s = lax.axis_index("s")
    wid = c * _NS + s
    _zero_rows(r0, 8)
    _zero_acc(acc_sh, r0, s)
    plsc.subcore_barrier()
    _edge_sweeps(u_hbm, acc_sh, epk, wid,
                 ebuf, (r0, r1, r2),
                 (g0, g1, g2), (s0, s1, s2), 8)
    plsc.subcore_barrier()

    @pl.when(c == 0)
    def _():
        _writeout(acc_sh, b0, r0, s)

    @pl.when(c == 1)
    def _():
        _writeout(acc_sh, b1, r0, s)


def _acc2_call(u2, epk):
    mesh = plsc.VectorSubcoreMesh(core_axis_name="c", subcore_axis_name="s")
    f = pl.kernel(
        _acc2_body,
        out_type=[jax.ShapeDtypeStruct((_NPAD, 128), jnp.float32)] * 2,
        mesh=mesh,
        scratch_types=[
            pltpu.VMEM((3, _GR, _CH), jnp.int32),
            pltpu.VMEM((_CH, 128), jnp.float32),
            pltpu.VMEM((_CH, 128), jnp.float32),
            pltpu.VMEM((_CH, 128), jnp.float32),
            pltpu.VMEM_SHARED((_NPAD, 128), jnp.float32),
            pltpu.SemaphoreType.DMA,
            pltpu.SemaphoreType.DMA,
            pltpu.SemaphoreType.DMA,
            pltpu.SemaphoreType.DMA,
            pltpu.SemaphoreType.DMA,
            pltpu.SemaphoreType.DMA,
        ],
    )
    return f(u2, epk)


def _mlp_body(x_ref, sty_ref, dga_ref, dgb_ref,
              w1, b1, ws1, bs1, w2, b2, ws2, bs2,
              w3, b3, ws3, bs3, w4, b4, ws4, bs4,
              h_ref, u0_ref, u1_ref, u2_ref, u3_ref):
    h = x_ref[0]
    sty = sty_ref[0]

    def unit(h, W, b, Ws, bs):
        d = W.shape[1]
        y = jnp.dot(h, W, preferred_element_type=jnp.float32) + b[0]
        mu = jnp.mean(y, axis=0, keepdims=True)
        var = jnp.mean((y - mu) * (y - mu), axis=0, keepdims=True)
        yn = (y - mu) / jnp.sqrt(var + 1e-5)
        gb = jnp.dot(sty, Ws, preferred_element_type=jnp.float32) + bs[0]
        gamma = gb[:, :d]
        beta = gb[:, d:]
        return _leaky((1.0 + gamma) * yn + beta)

    h = unit(h, w1[...], b1[...], ws1[...], bs1[...])
    h = unit(h, w2[...], b2[...], ws2[...], bs2[...])
    h = unit(h, w3[...], b3[...], ws3[...], bs3[...])
    h = unit(h, w4[...], b4[...], ws4[...], bs4[...])
    h_ref[0] = h
    dis = _dis_of(dga_ref[0] + dgb_ref[0])
    u = dis * h
    u0_ref[0] = u[:, 0:128]
    u1_ref[0] = u[:, 128:256]
    u2_ref[0] = u[:, 256:384]
    u3_ref[0] = u[:, 384:512]


def _mlp_call(x, sty, dga, dgb, params):
    N, D = x.shape
    Bn = sty.shape[0]
    R = N // Bn
    full = lambda shp: pl.BlockSpec(shp, lambda i: (0,) * len(shp))
    blk3 = lambda r, d: pl.BlockSpec((1, r, d), lambda i: (i, 0, 0))
    w_args = []
    w_specs = []
    for li in range(1, 5):
        W = params['fc%d_W' % li]
        b = params['fc%d_b' % li].reshape(1, -1)
        Ws = params['fc%d_Ws' % li]
        bs = params['fc%d_bs' % li].reshape(1, -1)
        w_args += [W, b, Ws, bs]
        w_specs += [full(W.shape), full(b.shape), full(Ws.shape), full(bs.shape)]
    out_shape = [jax.ShapeDtypeStruct((Bn, R, 512), jnp.float32)] + [
        jax.ShapeDtypeStruct((Bn, R, 128), jnp.float32)] * 4
    in_specs = [blk3(R, D), pl.BlockSpec((1, 1, sty.shape[1]), lambda i: (i, 0, 0)),
                blk3(R, 1), blk3(R, 1)] + w_specs
    out_specs = [blk3(R, 512)] + [blk3(R, 128)] * 4
    outs = pl.pallas_call(
        _mlp_body, grid=(Bn,), in_specs=in_specs, out_specs=out_specs,
        out_shape=out_shape)(
            x.reshape(Bn, R, D), sty.reshape(Bn, 1, -1),
            dga.reshape(Bn, R, 1), dgb.reshape(Bn, R, 1), *w_args)
    return tuple(o.reshape(N, -1) for o in outs)


def _conv_mm_body(h_ref, a0_ref, a1_ref, a2_ref, a3_ref,
                  dga_ref, dgb_ref, w0_ref, w1_ref, b_ref,
                  y_ref, st_ref):
    dis = _dis_of(dga_ref[0] + dgb_ref[0])
    ax = dis * jnp.concatenate(
        [a0_ref[0], a1_ref[0], a2_ref[0], a3_ref[0]], axis=1)
    y = (jnp.dot(h_ref[0], w0_ref[...], preferred_element_type=jnp.float32)
         - jnp.dot(ax, w1_ref[...], preferred_element_type=jnp.float32)
         + b_ref[0])
    y_ref[0] = y
    ps = jnp.sum(y, axis=0, keepdims=True)
    pq = jnp.sum(y * y, axis=0, keepdims=True)
    st_ref[...] = jnp.concatenate([ps[None], pq[None]], axis=1)


def _conv_mm_call(h, panels, dga, dgb, W0, W1, b):
    N, Din = h.shape
    Dout = W0.shape[1]
    R = 500
    G = N // R
    full = lambda shp: pl.BlockSpec(shp, lambda i: (0,) * len(shp))
    blk3 = lambda r, d: pl.BlockSpec((1, r, d), lambda i: (i, 0, 0))
    y, st = pl.pallas_call(
        _conv_mm_body, grid=(G,),
        in_specs=[blk3(R, Din)] + [blk3(R, 128)] * 4 +
                 [blk3(R, 1), blk3(R, 1),
                  full(W0.shape), full(W1.shape), full((1, Dout))],
        out_specs=[blk3(R, Dout),
                   pl.BlockSpec((1, 2, Dout), lambda i: (i, 0, 0))],
        out_shape=[
            jax.ShapeDtypeStruct((G, R, Dout), jnp.float32),
            jax.ShapeDtypeStruct((G, 2, Dout), jnp.float32),
        ])(h.reshape(G, R, Din),
           *[p[:N].reshape(G, R, 128) for p in panels],
           dga.reshape(G, R, 1), dgb.reshape(G, R, 1), W0, W1, b.reshape(1, -1))
    return y.reshape(N, Dout), st


def _norm_body_with_next(y_ref, st_ref, dga_ref, dgb_ref, g_ref, bb_ref,
                         wn_ref, h1_ref, un_ref, *, n_rows):
    st = jnp.sum(st_ref[...], axis=0)
    mu = st[0:1] / n_rows
    msq = st[1:2] / n_rows
    var = msq - mu * mu
    z = (y_ref[0] - mu) / jnp.sqrt(var + 1e-5) * g_ref[0] + bb_ref[0]
    h1 = _leaky(z)
    h1_ref[0] = h1
    dis = _dis_of(dga_ref[0] + dgb_ref[0])
    un_ref[0] = dis * jnp.dot(h1, wn_ref[...], preferred_element_type=jnp.float32)


def _norm_next_call(y, st, dga, dgb, g, bb, Wn):
    N, Dout = y.shape
    R = 500
    G = N // R
    Dn = Wn.shape[1]
    full = lambda shp: pl.BlockSpec(shp, lambda i: (0,) * len(shp))
    blk3 = lambda r, d: pl.BlockSpec((1, r, d), lambda i: (i, 0, 0))
    body = functools.partial(_norm_body_with_next, n_rows=float(N))
    h1, un = pl.pallas_call(
        body, grid=(G,),
        in_specs=[blk3(R, Dout), full(st.shape), blk3(R, 1), blk3(R, 1),
                  full((1, Dout)), full((1, Dout)), full(Wn.shape)],
        out_specs=[blk3(R, Dout), blk3(R, Dn)],
        out_shape=[
            jax.ShapeDtypeStruct((G, R, Dout), jnp.float32),
            jax.ShapeDtypeStruct((G, R, Dn), jnp.float32),
        ])(y.reshape(G, R, Dout), st, dga.reshape(G, R, 1),
           dgb.reshape(G, R, 1), g.reshape(1, -1), bb.reshape(1, -1), Wn)
    return h1.reshape(N, Dout), un.reshape(N, Dn)


def _norm_body_final(y_ref, st_ref, g_ref, bb_ref, o_ref, *, n_rows):
    st = jnp.sum(st_ref[...], axis=0)
    mu = st[0:1] / n_rows
    msq = st[1:2] / n_rows
    var = msq - mu * mu
    z = (y_ref[0] - mu) / jnp.sqrt(var + 1e-5) * g_ref[0] + bb_ref[0]
    o_ref[0] = _leaky(z)


def _norm_final_call(y, st, g, bb):
    N, Dout = y.shape
    R = 500
    G = N // R
    full = lambda shp: pl.BlockSpec(shp, lambda i: (0,) * len(shp))
    blk3 = lambda r, d: pl.BlockSpec((1, r, d), lambda i: (i, 0, 0))
    body = functools.partial(_norm_body_final, n_rows=float(N))
    out = pl.pallas_call(
        body, grid=(G,),
        in_specs=[blk3(R, Dout), full(st.shape), full((1, Dout)), full((1, Dout))],
        out_specs=blk3(R, Dout),
        out_shape=jax.ShapeDtypeStruct((G, R, Dout), jnp.float32),
    )(y.reshape(G, R, Dout), st, g.reshape(1, -1), bb.reshape(1, -1))
    return out.reshape(N, Dout)


def _conv2_mm_body(h_ref, aa_ref, ab_ref, dga_ref, dgb_ref, w0_ref, b_ref,
                   y_ref, st_ref):
    dis = _dis_of(dga_ref[0] + dgb_ref[0])
    d = w0_ref.shape[1]
    ax = dis * (aa_ref[0][:, :d] + ab_ref[0][:, :d])
    y = (jnp.dot(h_ref[0], w0_ref[...], preferred_element_type=jnp.float32)
         - ax + b_ref[0])
    y_ref[0] = y
    ps = jnp.sum(y, axis=0, keepdims=True)
    pq = jnp.sum(y * y, axis=0, keepdims=True)
    st_ref[...] = jnp.concatenate([ps[None], pq[None]], axis=1)


def _conv2_mm_call(h1, acc2a, acc2b, dga, dgb, W0, b):
    N, Din = h1.shape
    Dout = W0.shape[1]
    R = 500
    G = N // R
    full = lambda shp: pl.BlockSpec(shp, lambda i: (0,) * len(shp))
    blk3 = lambda r, d: pl.BlockSpec((1, r, d), lambda i: (i, 0, 0))
    y, st = pl.pallas_call(
        _conv2_mm_body, grid=(G,),
        in_specs=[blk3(R, Din), blk3(R, 128), blk3(R, 128),
                  blk3(R, 1), blk3(R, 1), full(W0.shape), full((1, Dout))],
        out_specs=[blk3(R, Dout),
                   pl.BlockSpec((1, 2, Dout), lambda i: (i, 0, 0))],
        out_shape=[
            jax.ShapeDtypeStruct((G, R, Dout), jnp.float32),
            jax.ShapeDtypeStruct((G, 2, Dout), jnp.float32),
        ])(h1.reshape(G, R, Din), acc2a.reshape(G, R, 128),
           acc2b.reshape(G, R, 128), dga.reshape(G, R, 1),
           dgb.reshape(G, R, 1), W0, b.reshape(1, -1))
    return y.reshape(N, Dout), st


def kernel(x, style_vector, edge_index, edge_attr, batch_size, nroi, params):
    N, D = x.shape
    Bn = style_vector.shape[0]
    R = N // Bn

    E = edge_index.shape[1]
    src = edge_index[0].astype(jnp.int32)
    dst = edge_index[1].astype(jnp.int32)
    ew = edge_attr.astype(jnp.float32)
    nw = _NC * _NS
    gsz = _GR * _CH
    ewb = jax.lax.bitcast_convert_type(ew, jnp.int32)
    epk16 = jnp.stack([src.reshape(_NS, E // (_NS * gsz), _GR, _CH),
                       dst.reshape(_NS, E // (_NS * gsz), _GR, _CH),
                       ewb.reshape(_NS, E // (_NS * gsz), _GR, _CH)], axis=2)
    epk32 = jnp.stack([src.reshape(nw, E // (nw * gsz), _GR, _CH),
                       dst.reshape(nw, E // (nw * gsz), _GR, _CH),
                       ewb.reshape(nw, E // (nw * gsz), _GR, _CH)], axis=2)
    srcd = src.reshape(nw, E // (nw * _CH), _CH)
    ewd = ew.reshape(nw, E // (nw * _CH), _CH)

    dg0, dg1 = _deg_call(srcd, ewd)
    dga = dg0[:N].reshape(N, 1)
    dgb = dg1[:N].reshape(N, 1)

    h, u0, u1, u2, u3 = _mlp_call(x, style_vector, dga, dgb, params)

    a0, a1, a2, a3 = _acc1_call((u0, u1, u2, u3), epk16)

    y1, st1 = _conv_mm_call(h, (a0, a1, a2, a3), dga, dgb,
                            params['conv1_W0'], params['conv1_W1'],
                            params['conv1_b'])
    w1n = jnp.pad(params['conv2_W1'], ((0, 0), (0, 64)))
    h1, u2w = _norm_next_call(y1, st1, dga, dgb,
                              params['conv1_g'], params['conv1_bb'], w1n)

    b0, b1 = _acc2_call(u2w, epk32)

    y2, st2 = _conv2_mm_call(h1, b0[:N], b1[:N], dga, dgb,
                             params['conv2_W0'], params['conv2_b'])
    out = _norm_final_call(y2, st2, params['conv2_g'], params['conv2_bb'])
    return out.reshape(Bn, R, out.shape[1])
